# baseline (speedup 1.0000x reference)
import dataclasses
import functools
import math

import jax
import jax.numpy as jnp
from jax import lax
from jax.experimental import pallas as pl
from jax.experimental.pallas import tpu as pltpu
from jax.experimental.pallas import tpu_sc as plsc

N_SC_CORES = 2
N_SUBCORES = 16
N_WORKERS = N_SC_CORES * N_SUBCORES
NUM_BATCHES = 16
NEG_INF = -1e9


def _mm_body(a_ref, b_ref, o_ref):
    o_ref[...] = jax.lax.dot_general(
        a_ref[...], b_ref[...], (((1,), (0,)), ((), ())),
        preferred_element_type=jnp.float32).astype(o_ref.dtype)


def _mm_bias_body(a_ref, b_ref, bias_ref, o_ref):
    o_ref[...] = (jax.lax.dot_general(
        a_ref[...], b_ref[...], (((1,), (0,)), ((), ())),
        preferred_element_type=jnp.float32)
        + bias_ref[...]).astype(o_ref.dtype)


def _mm2_body(a1_ref, b1_ref, a2_ref, b2_ref, o_ref):
    o_ref[...] = (jax.lax.dot_general(
        a1_ref[...], b1_ref[...], (((1,), (0,)), ((), ())),
        preferred_element_type=jnp.float32) + jax.lax.dot_general(
        a2_ref[...], b2_ref[...], (((1,), (0,)), ((), ())),
        preferred_element_type=jnp.float32)).astype(o_ref.dtype)


def _mm2(a1, b1, a2, b2, bm=512, bn=512, out_dtype=jnp.float32):
    M, K1 = a1.shape
    _, N = b1.shape
    K2 = a2.shape[1]
    bm = min(bm, M)
    bn = min(bn, N)
    grid = (M // bm, N // bn)
    return pl.pallas_call(
        _mm2_body,
        grid=grid,
        in_specs=[
            pl.BlockSpec((bm, K1), lambda i, j: (i, 0)),
            pl.BlockSpec((K1, bn), lambda i, j: (0, j)),
            pl.BlockSpec((bm, K2), lambda i, j: (i, 0)),
            pl.BlockSpec((K2, bn), lambda i, j: (0, j)),
        ],
        out_specs=pl.BlockSpec((bm, bn), lambda i, j: (i, j)),
        out_shape=jax.ShapeDtypeStruct((M, N), out_dtype),
        compiler_params=pltpu.CompilerParams(
            dimension_semantics=("parallel", "parallel")),
    )(a1, b1, a2, b2)


def _matmul(a, b, bias=None, bm=512, bn=512, out_dtype=jnp.float32):
    M, K = a.shape
    _, N = b.shape
    bm = min(bm, M)
    bn = min(bn, N)
    grid = (M // bm, N // bn)
    in_specs = [
        pl.BlockSpec((bm, K), lambda i, j: (i, 0)),
        pl.BlockSpec((K, bn), lambda i, j: (0, j)),
    ]
    args = [a, b]
    body = _mm_body
    if bias is not None:
        in_specs.append(pl.BlockSpec((1, bn), lambda i, j: (0, j)))
        args.append(bias.reshape(1, N))
        body = _mm_bias_body
    return pl.pallas_call(
        body,
        grid=grid,
        in_specs=in_specs,
        out_specs=pl.BlockSpec((bm, bn), lambda i, j: (i, j)),
        out_shape=jax.ShapeDtypeStruct((M, N), out_dtype),
        compiler_params=pltpu.CompilerParams(
            dimension_semantics=("parallel", "parallel")),
    )(*args)


def _sc_gather_rows(table, idx):
    n = idx.shape[0]
    _, D = table.shape
    per_w = n // N_WORKERS
    chunk = min(per_w, 64)
    nchunks = per_w // chunk
    mesh = plsc.VectorSubcoreMesh(core_axis_name="c", subcore_axis_name="s")

    @functools.partial(
        pl.kernel, mesh=mesh,
        out_type=jax.ShapeDtypeStruct((n, D), jnp.float32),
        scratch_types=[
            pltpu.VMEM((chunk,), jnp.int32),
            pltpu.VMEM((chunk, D), jnp.float32),
            pltpu.SemaphoreType.DMA,
        ])
    def k(table_hbm, idx_hbm, out_hbm, idx_v, rows_v, sem):
        wid = lax.axis_index("s") * N_SC_CORES + lax.axis_index("c")
        base = wid * per_w

        @pl.loop(0, nchunks)
        def _(c):
            b = base + c * chunk
            pltpu.sync_copy(idx_hbm.at[pl.ds(b, chunk)], idx_v)
            pltpu.async_copy(table_hbm.at[idx_v], rows_v, sem).wait()
            pltpu.sync_copy(rows_v, out_hbm.at[pl.ds(b, chunk)])

    return k(table, idx)


def _sc_build_adj(src, dst, etype, ny):
    E = src.shape[0]
    per_rows = ny // N_WORKERS
    half = per_rows // 2
    mesh = plsc.VectorSubcoreMesh(core_axis_name="c", subcore_axis_name="s")

    cp = pltpu.CompilerParams()
    if "needs_layout_passes" in pltpu.CompilerParams.__dataclass_fields__:
        cp = dataclasses.replace(cp, needs_layout_passes=False)

    @functools.partial(
        pl.kernel, mesh=mesh, compiler_params=cp,
        out_type=(jax.ShapeDtypeStruct((ny, ny), jnp.float32),
                  jax.ShapeDtypeStruct((ny, 16), jnp.float32)),
        scratch_types=[
            pltpu.VMEM((E,), jnp.int32),
            pltpu.VMEM((E,), jnp.int32),
            pltpu.VMEM((E,), jnp.int32),
            pltpu.VMEM((E + 16,), jnp.int32),
            pltpu.VMEM((E + 16,), jnp.int32),
            pltpu.VMEM((E + 16,), jnp.int32),
            pltpu.VMEM((half, ny), jnp.float32),
            pltpu.VMEM((per_rows, 16), jnp.float32),
        ])
    def k(src_hbm, dst_hbm, et_hbm, adj_hbm, cnt_hbm,
          src_v, dst_v, et_v, cs_v, cd_v, ct_v, acc_v, c_v):
        cid = lax.axis_index("c")
        sid = lax.axis_index("s")
        wid = sid * N_SC_CORES + cid
        row0 = wid * per_rows
        pltpu.sync_copy(src_hbm, src_v)
        pltpu.sync_copy(dst_hbm, dst_v)
        pltpu.sync_copy(et_hbm, et_v)

        @pl.loop(0, per_rows)
        def _(r):
            c_v.at[r, pl.ds(0, 16)][...] = jnp.zeros((16,), jnp.float32)

        lanes = lax.iota(jnp.int32, 16)
        ones = jnp.full((16,), 1.0, jnp.float32)

        def compress(g, off):
            b = g * 16
            d_v = dst_v[pl.ds(b, 16)]
            m = (d_v >= row0) & (d_v < row0 + per_rows)
            plsc.store_compressed(cd_v.at[pl.ds(off, 16)], d_v, mask=m)
            plsc.store_compressed(cs_v.at[pl.ds(off, 16)],
                                  src_v[pl.ds(b, 16)], mask=m)
            plsc.store_compressed(ct_v.at[pl.ds(off, 16)],
                                  et_v[pl.ds(b, 16)], mask=m)
            return off + jnp.sum(m.astype(jnp.int32), axis=0)

        nmine = lax.fori_loop(0, E // 16, compress, jnp.int32(0))
        ng = (nmine + 15) // 16

        def cnt_pass(g, _):
            b = g * 16
            valid = lanes < (nmine - b)
            d_v = cd_v[pl.ds(b, 16)]
            t_v = ct_v[pl.ds(b, 16)]
            rc_v = jnp.clip(d_v - row0, 0, per_rows - 1)
            for l in range(16):
                plsc.addupdate_scatter(
                    c_v, [rc_v, t_v], ones, mask=valid & (lanes == l))
            return 0

        lax.fori_loop(0, ng, cnt_pass, 0)

        for p in range(2):
            plo = row0 + p * half

            @pl.loop(0, half)
            def _(r):
                @pl.loop(0, ny, step=16)
                def _(cc):
                    acc_v.at[r, pl.ds(cc, 16)][...] = jnp.zeros(
                        (16,), jnp.float32)

            def adj_pass(g, _):
                b = g * 16
                valid = lanes < (nmine - b)
                d_v = cd_v[pl.ds(b, 16)]
                s_v = cs_v[pl.ds(b, 16)]
                in_a = valid & (d_v >= plo) & (d_v < plo + half)
                r_v = jnp.clip(d_v - plo, 0, half - 1)
                for l in range(16):
                    plsc.addupdate_scatter(
                        acc_v, [r_v, s_v], ones, mask=in_a & (lanes == l))
                return 0

            lax.fori_loop(0, ng, adj_pass, 0)
            pltpu.sync_copy(acc_v, adj_hbm.at[pl.ds(plo, half)])

        pltpu.sync_copy(c_v, cnt_hbm.at[pl.ds(row0, per_rows)])

    return k(src, dst, etype)


def _sc_edge_scores(ab, src, dst):
    E = src.shape[0]
    W = ab.shape[1]
    per_w = E // N_WORKERS
    chunk = min(per_w, 64)
    nchunks = per_w // chunk
    mesh = plsc.VectorSubcoreMesh(core_axis_name="c", subcore_axis_name="s")

    cp = pltpu.CompilerParams()
    if "needs_layout_passes" in pltpu.CompilerParams.__dataclass_fields__:
        cp = dataclasses.replace(cp, needs_layout_passes=False)

    @functools.partial(
        pl.kernel, mesh=mesh, compiler_params=cp,
        out_type=jax.ShapeDtypeStruct((E, 16), jnp.float32),
        scratch_types=[
            pltpu.VMEM((chunk,), jnp.int32),
            pltpu.VMEM((chunk,), jnp.int32),
            pltpu.VMEM((chunk, W), jnp.float32),
            pltpu.VMEM((chunk, W), jnp.float32),
            pltpu.VMEM((chunk, 16), jnp.float32),
            pltpu.SemaphoreType.DMA,
            pltpu.SemaphoreType.DMA,
        ])
    def k(ab_hbm, src_hbm, dst_hbm, out_hbm,
          src_v, dst_v, arow_v, brow_v, o_v, sem1, sem2):
        wid = lax.axis_index("s") * N_SC_CORES + lax.axis_index("c")
        base = wid * per_w
        lanes = lax.iota(jnp.int32, 16)

        @pl.loop(0, nchunks)
        def _(c):
            b = base + c * chunk
            pltpu.sync_copy(src_hbm.at[pl.ds(b, chunk)], src_v)
            pltpu.sync_copy(dst_hbm.at[pl.ds(b, chunk)], dst_v)
            cp1 = pltpu.async_copy(ab_hbm.at[src_v], arow_v, sem1)
            cp2 = pltpu.async_copy(ab_hbm.at[dst_v], brow_v, sem2)
            cp1.wait()
            cp2.wait()

            @pl.loop(0, chunk)
            def _(i):
                row = jnp.full((16,), 0, jnp.int32) + i
                a = plsc.load_gather(arow_v, [row, lanes])
                bb = plsc.load_gather(brow_v, [row, lanes + 16])
                plsc.store_scatter(o_v, [row, lanes], a + bb)

            pltpu.sync_copy(o_v, out_hbm.at[pl.ds(b, chunk)])

    return k(ab, src, dst)


def _flash_layer(s_arr, s_col, agg, wq, kv, k_col, v_col,
                 yb3, xb3, sched, D, BY=128, BX=512,
                 out_dtype=jnp.float32):
    NYr = agg.shape[0]
    NXr = kv.shape[0]
    ny = NYr // BY
    nx = NXr // BX
    scale = 1.0 / math.sqrt(float(D))
    kb_col = k_col // D
    vb_col = v_col // D
    sb_col = s_col // D

    def body(sched_ref, s_ref, a_ref, wq_ref, k_ref, v_ref,
             yb_ref, xb_ref, o_ref, h_s, q_s, acc_s, m_s, l_s):
        i = pl.program_id(0)
        j = pl.program_id(1)

        @pl.when(j == 0)
        def _():
            h = s_ref[...] + a_ref[...]
            h_s[...] = h
            q_s[...] = (jnp.dot(h.astype(jnp.bfloat16), wq_ref[...],
                                preferred_element_type=jnp.float32)
                        * scale).astype(jnp.bfloat16)
            acc_s[...] = jnp.zeros_like(acc_s)
            m_s[...] = jnp.full_like(m_s, -jnp.inf)
            l_s[...] = jnp.zeros_like(l_s)

        lo = sched_ref[0, i]
        hi = sched_ref[1, i]

        @pl.when((j >= lo) & (j <= hi))
        def _():
            sc = lax.dot_general(
                q_s[...], k_ref[...], (((1,), (1,)), ((), ())),
                preferred_element_type=jnp.float32)
            yb = yb_ref[0]
            xb = xb_ref[0]
            mask = yb == xb
            sc = jnp.where(mask, sc, NEG_INF)
            m_prev = m_s[...]
            m_new = jnp.maximum(m_prev, jnp.max(sc, axis=1, keepdims=True))
            p = jnp.exp(sc - m_new)
            corr = jnp.exp(m_prev - m_new)
            l_s[...] = l_s[...] * corr + jnp.sum(p, axis=1, keepdims=True)
            m_s[...] = m_new
            acc_s[...] = acc_s[...] * corr + jnp.dot(
                p.astype(jnp.bfloat16), v_ref[...],
                preferred_element_type=jnp.float32)

        @pl.when(j == nx - 1)
        def _():
            o_ref[...] = jnp.maximum(
                h_s[...] + acc_s[...] / l_s[...], 0.0).astype(o_ref.dtype)

    def kv_index(col):
        def f(i, j, sched):
            return (jnp.clip(j, sched[0, i], sched[1, i]), col)
        return f

    def xb_index(i, j, sched):
        return (jnp.clip(j, sched[0, i], sched[1, i]), 0, 0)

    grid_spec = pltpu.PrefetchScalarGridSpec(
        num_scalar_prefetch=1,
        grid=(ny, nx),
        in_specs=[
            pl.BlockSpec((BY, D), lambda i, j, sched: (i, sb_col)),
            pl.BlockSpec((BY, D), lambda i, j, sched: (i, 0)),
            pl.BlockSpec((D, D), lambda i, j, sched: (0, 0)),
            pl.BlockSpec((BX, D), kv_index(kb_col)),
            pl.BlockSpec((BX, D), kv_index(vb_col)),
            pl.BlockSpec((1, BY, 1), lambda i, j, sched: (i, 0, 0)),
            pl.BlockSpec((1, 1, BX), xb_index),
        ],
        out_specs=pl.BlockSpec((BY, D), lambda i, j, sched: (i, 0)),
        scratch_shapes=[
            pltpu.VMEM((BY, D), jnp.float32),
            pltpu.VMEM((BY, D), jnp.bfloat16),
            pltpu.VMEM((BY, D), jnp.float32),
            pltpu.VMEM((BY, 1), jnp.float32),
            pltpu.VMEM((BY, 1), jnp.float32),
        ],
    )
    return pl.pallas_call(
        body,
        grid_spec=grid_spec,
        out_shape=jax.ShapeDtypeStruct((NYr, D), out_dtype),
        compiler_params=pltpu.CompilerParams(
            dimension_semantics=("parallel", "arbitrary")),
    )(sched, s_arr, agg, wq, kv, kv, yb3, xb3)


def _block_schedule(y_batch, x_batch, BY, BX, nx):
    ny = y_batch.shape[0] // BY
    b_lo = y_batch[::BY]
    b_hi = y_batch[BY - 1::BY]
    bounds = jnp.searchsorted(x_batch, jnp.arange(NUM_BATCHES + 1),
                              side='left').astype(jnp.int32)
    xs = bounds[b_lo]
    xe = bounds[b_hi + 1]
    lo = jnp.clip(xs // BX, 0, nx - 1)
    hi = jnp.clip(jnp.maximum((xe - 1) // BX, lo), 0, nx - 1)
    return jnp.stack([lo, hi]).astype(jnp.int32)


def kernel(x, x_batch, tgt_y, tgt_edge_index, tgt_edge_type, tgt_y_batch,
           params):
    p = params
    g1, g2, g3 = p['gcn1'], p['gcn2'], p['gcn3']
    src = tgt_edge_index[0]
    dst = tgt_edge_index[1]
    H1 = g1['W_self'].shape[1]
    H2 = g2['W_self'].shape[1]
    H3 = g3['W_self'].shape[1]

    BY, BX = 128, 512
    NXr = x.shape[0]
    nx = NXr // BX
    ny = tgt_y_batch.shape[0] // BY
    sched = _block_schedule(tgt_y_batch, x_batch, BY, BX, nx)
    yb3 = tgt_y_batch.reshape(ny, BY, 1)
    xb3 = x_batch.reshape(nx, 1, BX)

    bf16 = jnp.bfloat16

    kvw = jnp.concatenate(
        [g1['Wk'], g1['Wv'], g2['Wk'], g2['Wv'], g3['Wk'], g3['Wv']], axis=1)
    kv = _matmul(x.astype(bf16), kvw.astype(bf16), out_dtype=bf16, bm=2048)

    ny_nodes = tgt_y.shape[0]
    adj, cnt = _sc_build_adj(src, dst, tgt_edge_type, ny_nodes)
    adjb = adj.astype(bf16)
    cntb = cnt.astype(bf16)

    def _emb_pad(emb):
        return jnp.zeros((16, emb.shape[1]), bf16).at[
            :emb.shape[0]].set(emb.astype(bf16))

    y0 = _sc_gather_rows(p['embed'], tgt_y)
    y0b = y0.astype(bf16)
    s1 = _matmul(y0b, g1['W_self'].astype(bf16))
    ay0 = _matmul(adjb, y0b, out_dtype=bf16)
    agg1 = _mm2(ay0, g1['W_nb'].astype(bf16), cntb, _emb_pad(g1['edge_emb']))
    y1 = _flash_layer(s1, 0, agg1, g1['Wq'].astype(bf16), kv, 0, H1,
                      yb3, xb3, sched, H1, BY, BX, out_dtype=bf16)

    s2 = _matmul(y1, g2['W_self'].astype(bf16))
    m2 = _matmul(y1, g2['W_nb'].astype(bf16), out_dtype=bf16)
    agg2 = _mm2(adjb, m2, cntb, _emb_pad(g2['edge_emb']))
    y2 = _flash_layer(s2, 0, agg2, g2['Wq'].astype(bf16), kv,
                      2 * H1, 2 * H1 + H2, yb3, xb3, sched, H2, BY, BX,
                      out_dtype=bf16)

    s3 = _matmul(y2, g3['W_self'].astype(bf16))
    m3 = _matmul(y2, g3['W_nb'].astype(bf16), out_dtype=bf16)
    agg3 = _mm2(adjb, m3, cntb, _emb_pad(g3['edge_emb']))
    y3 = _flash_layer(s3, 0, agg3, g3['Wq'].astype(bf16), kv,
                      2 * (H1 + H2), 2 * (H1 + H2) + H3,
                      yb3, xb3, sched, H3, BY, BX)

    y3b = y3.astype(bf16)
    y_score = _matmul(y3b, p['Wz'].astype(bf16), bias=p['bz'])

    emb_d = y3.shape[1]
    n_rel = p['Wg'].shape[1]
    wg_pad = jnp.zeros((emb_d, 128), jnp.float32)
    wg_pad = wg_pad.at[:, 0:n_rel].set(p['Wg'][:emb_d])
    wg_pad = wg_pad.at[:, 16:16 + n_rel].set(p['Wg'][emb_d:])
    bg_pad = jnp.zeros((128,), jnp.float32).at[0:n_rel].set(p['bg'])
    ab = _matmul(y3b, wg_pad.astype(bf16), bias=bg_pad, bn=128)
    er = _sc_edge_scores(ab, src, dst)
    y_edge_rel_score = lax.slice_in_dim(er, 0, n_rel, axis=1)

    return (y3, tgt_y_batch, tgt_edge_index, tgt_edge_type, y_score,
            y_edge_rel_score)


# speedup vs baseline: 1.2112x; 1.2112x over previous
import dataclasses
import functools
import math

import jax
import jax.numpy as jnp
from jax import lax
from jax.experimental import pallas as pl
from jax.experimental.pallas import tpu as pltpu
from jax.experimental.pallas import tpu_sc as plsc

N_SC_CORES = 2
N_SUBCORES = 16
N_WORKERS = N_SC_CORES * N_SUBCORES
NUM_BATCHES = 16
NEG_INF = -1e9


def _mm_body(a_ref, b_ref, o_ref):
    o_ref[...] = jax.lax.dot_general(
        a_ref[...], b_ref[...], (((1,), (0,)), ((), ())),
        preferred_element_type=jnp.float32).astype(o_ref.dtype)


def _mm_bias_body(a_ref, b_ref, bias_ref, o_ref):
    o_ref[...] = (jax.lax.dot_general(
        a_ref[...], b_ref[...], (((1,), (0,)), ((), ())),
        preferred_element_type=jnp.float32)
        + bias_ref[...]).astype(o_ref.dtype)


def _mm2_body(a1_ref, b1_ref, a2_ref, b2_ref, o_ref):
    o_ref[...] = (jax.lax.dot_general(
        a1_ref[...], b1_ref[...], (((1,), (0,)), ((), ())),
        preferred_element_type=jnp.float32) + jax.lax.dot_general(
        a2_ref[...], b2_ref[...], (((1,), (0,)), ((), ())),
        preferred_element_type=jnp.float32)).astype(o_ref.dtype)


def _mm2(a1, b1, a2, b2, bm=512, bn=512, out_dtype=jnp.float32):
    M, K1 = a1.shape
    _, N = b1.shape
    K2 = a2.shape[1]
    bm = min(bm, M)
    bn = min(bn, N)
    grid = (M // bm, N // bn)
    return pl.pallas_call(
        _mm2_body,
        grid=grid,
        in_specs=[
            pl.BlockSpec((bm, K1), lambda i, j: (i, 0)),
            pl.BlockSpec((K1, bn), lambda i, j: (0, j)),
            pl.BlockSpec((bm, K2), lambda i, j: (i, 0)),
            pl.BlockSpec((K2, bn), lambda i, j: (0, j)),
        ],
        out_specs=pl.BlockSpec((bm, bn), lambda i, j: (i, j)),
        out_shape=jax.ShapeDtypeStruct((M, N), out_dtype),
        compiler_params=pltpu.CompilerParams(
            dimension_semantics=("parallel", "parallel")),
    )(a1, b1, a2, b2)


def _matmul(a, b, bias=None, bm=512, bn=512, out_dtype=jnp.float32):
    M, K = a.shape
    _, N = b.shape
    bm = min(bm, M)
    bn = min(bn, N)
    grid = (M // bm, N // bn)
    in_specs = [
        pl.BlockSpec((bm, K), lambda i, j: (i, 0)),
        pl.BlockSpec((K, bn), lambda i, j: (0, j)),
    ]
    args = [a, b]
    body = _mm_body
    if bias is not None:
        in_specs.append(pl.BlockSpec((1, bn), lambda i, j: (0, j)))
        args.append(bias.reshape(1, N))
        body = _mm_bias_body
    return pl.pallas_call(
        body,
        grid=grid,
        in_specs=in_specs,
        out_specs=pl.BlockSpec((bm, bn), lambda i, j: (i, j)),
        out_shape=jax.ShapeDtypeStruct((M, N), out_dtype),
        compiler_params=pltpu.CompilerParams(
            dimension_semantics=("parallel", "parallel")),
    )(*args)


def _sc_gather_rows(table, idx):
    n = idx.shape[0]
    _, D = table.shape
    per_w = n // N_WORKERS
    chunk = min(per_w, 64)
    nchunks = per_w // chunk
    mesh = plsc.VectorSubcoreMesh(core_axis_name="c", subcore_axis_name="s")

    @functools.partial(
        pl.kernel, mesh=mesh,
        out_type=jax.ShapeDtypeStruct((n, D), jnp.float32),
        scratch_types=[
            pltpu.VMEM((chunk,), jnp.int32),
            pltpu.VMEM((chunk, D), jnp.float32),
            pltpu.SemaphoreType.DMA,
        ])
    def k(table_hbm, idx_hbm, out_hbm, idx_v, rows_v, sem):
        wid = lax.axis_index("s") * N_SC_CORES + lax.axis_index("c")
        base = wid * per_w

        @pl.loop(0, nchunks)
        def _(c):
            b = base + c * chunk
            pltpu.sync_copy(idx_hbm.at[pl.ds(b, chunk)], idx_v)
            pltpu.async_copy(table_hbm.at[idx_v], rows_v, sem).wait()
            pltpu.sync_copy(rows_v, out_hbm.at[pl.ds(b, chunk)])

    return k(table, idx)


def _sc_build_adj(src, dst, etype, ny):
    E = src.shape[0]
    per_rows = ny // N_WORKERS
    half = per_rows // 2
    mesh = plsc.VectorSubcoreMesh(core_axis_name="c", subcore_axis_name="s")

    cp = pltpu.CompilerParams()
    if "needs_layout_passes" in pltpu.CompilerParams.__dataclass_fields__:
        cp = dataclasses.replace(cp, needs_layout_passes=False)

    @functools.partial(
        pl.kernel, mesh=mesh, compiler_params=cp,
        out_type=(jax.ShapeDtypeStruct((ny, ny), jnp.float32),
                  jax.ShapeDtypeStruct((ny, 16), jnp.float32)),
        scratch_types=[
            pltpu.VMEM((E,), jnp.int32),
            pltpu.VMEM((E,), jnp.int32),
            pltpu.VMEM((E,), jnp.int32),
            pltpu.VMEM((E + 16,), jnp.int32),
            pltpu.VMEM((E + 16,), jnp.int32),
            pltpu.VMEM((E + 16,), jnp.int32),
            pltpu.VMEM((half, ny), jnp.float32),
            pltpu.VMEM((per_rows, 16), jnp.float32),
        ])
    def k(src_hbm, dst_hbm, et_hbm, adj_hbm, cnt_hbm,
          src_v, dst_v, et_v, cs_v, cd_v, ct_v, acc_v, c_v):
        cid = lax.axis_index("c")
        sid = lax.axis_index("s")
        wid = sid * N_SC_CORES + cid
        row0 = wid * per_rows
        pltpu.sync_copy(src_hbm, src_v)
        pltpu.sync_copy(dst_hbm, dst_v)
        pltpu.sync_copy(et_hbm, et_v)

        @pl.loop(0, per_rows)
        def _(r):
            c_v.at[r, pl.ds(0, 16)][...] = jnp.zeros((16,), jnp.float32)

        lanes = lax.iota(jnp.int32, 16)
        ones = jnp.full((16,), 1.0, jnp.float32)

        def compress(g, off):
            b = g * 16
            d_v = dst_v[pl.ds(b, 16)]
            m = (d_v >= row0) & (d_v < row0 + per_rows)
            plsc.store_compressed(cd_v.at[pl.ds(off, 16)], d_v, mask=m)
            plsc.store_compressed(cs_v.at[pl.ds(off, 16)],
                                  src_v[pl.ds(b, 16)], mask=m)
            plsc.store_compressed(ct_v.at[pl.ds(off, 16)],
                                  et_v[pl.ds(b, 16)], mask=m)
            return off + jnp.sum(m.astype(jnp.int32), axis=0)

        nmine = lax.fori_loop(0, E // 16, compress, jnp.int32(0))
        ng = (nmine + 15) // 16

        def cnt_pass(g, _):
            b = g * 16
            valid = lanes < (nmine - b)
            d_v = cd_v[pl.ds(b, 16)]
            t_v = ct_v[pl.ds(b, 16)]
            rc_v = jnp.clip(d_v - row0, 0, per_rows - 1)
            for l in range(16):
                plsc.addupdate_scatter(
                    c_v, [rc_v, t_v], ones, mask=valid & (lanes == l))
            return 0

        lax.fori_loop(0, ng, cnt_pass, 0)

        for p in range(2):
            plo = row0 + p * half

            @pl.loop(0, half)
            def _(r):
                @pl.loop(0, ny, step=16)
                def _(cc):
                    acc_v.at[r, pl.ds(cc, 16)][...] = jnp.zeros(
                        (16,), jnp.float32)

            def adj_pass(g, _):
                b = g * 16
                valid = lanes < (nmine - b)
                d_v = cd_v[pl.ds(b, 16)]
                s_v = cs_v[pl.ds(b, 16)]
                in_a = valid & (d_v >= plo) & (d_v < plo + half)
                r_v = jnp.clip(d_v - plo, 0, half - 1)
                for l in range(16):
                    plsc.addupdate_scatter(
                        acc_v, [r_v, s_v], ones, mask=in_a & (lanes == l))
                return 0

            lax.fori_loop(0, ng, adj_pass, 0)
            pltpu.sync_copy(acc_v, adj_hbm.at[pl.ds(plo, half)])

        pltpu.sync_copy(c_v, cnt_hbm.at[pl.ds(row0, per_rows)])

    return k(src, dst, etype)


def _sc_edge_scores(ab, src, dst):
    E = src.shape[0]
    W = ab.shape[1]
    per_w = E // N_WORKERS
    chunk = min(per_w, 64)
    nchunks = per_w // chunk
    mesh = plsc.VectorSubcoreMesh(core_axis_name="c", subcore_axis_name="s")

    cp = pltpu.CompilerParams()
    if "needs_layout_passes" in pltpu.CompilerParams.__dataclass_fields__:
        cp = dataclasses.replace(cp, needs_layout_passes=False)

    @functools.partial(
        pl.kernel, mesh=mesh, compiler_params=cp,
        out_type=jax.ShapeDtypeStruct((E, 16), jnp.float32),
        scratch_types=[
            pltpu.VMEM((chunk,), jnp.int32),
            pltpu.VMEM((chunk,), jnp.int32),
            pltpu.VMEM((chunk, W), jnp.float32),
            pltpu.VMEM((chunk, W), jnp.float32),
            pltpu.VMEM((chunk, 16), jnp.float32),
            pltpu.SemaphoreType.DMA,
            pltpu.SemaphoreType.DMA,
        ])
    def k(ab_hbm, src_hbm, dst_hbm, out_hbm,
          src_v, dst_v, arow_v, brow_v, o_v, sem1, sem2):
        wid = lax.axis_index("s") * N_SC_CORES + lax.axis_index("c")
        base = wid * per_w
        lanes = lax.iota(jnp.int32, 16)

        @pl.loop(0, nchunks)
        def _(c):
            b = base + c * chunk
            pltpu.sync_copy(src_hbm.at[pl.ds(b, chunk)], src_v)
            pltpu.sync_copy(dst_hbm.at[pl.ds(b, chunk)], dst_v)
            cp1 = pltpu.async_copy(ab_hbm.at[src_v], arow_v, sem1)
            cp2 = pltpu.async_copy(ab_hbm.at[dst_v], brow_v, sem2)
            cp1.wait()
            cp2.wait()

            @pl.loop(0, chunk)
            def _(i):
                row = jnp.full((16,), 0, jnp.int32) + i
                a = plsc.load_gather(arow_v, [row, lanes])
                bb = plsc.load_gather(brow_v, [row, lanes + 16])
                plsc.store_scatter(o_v, [row, lanes], a + bb)

            pltpu.sync_copy(o_v, out_hbm.at[pl.ds(b, chunk)])

    return k(ab, src, dst)


def _flash_layer(s_arr, s_col, agg, wq, kv, k_col, v_col,
                 yb3, xb3, sched, D, BY=128, BX=512,
                 out_dtype=jnp.float32):
    NYr = agg.shape[0]
    NXr = kv.shape[0]
    ny = NYr // BY
    nx = NXr // BX
    scale = 1.0 / math.sqrt(float(D))
    kb_col = k_col // D
    vb_col = v_col // D
    sb_col = s_col // D

    def body(sched_ref, s_ref, a_ref, wq_ref, k_ref, v_ref,
             yb_ref, xb_ref, o_ref, h_s, q_s, acc_s, m_s, l_s):
        i = pl.program_id(0)
        j = pl.program_id(1)

        @pl.when(j == 0)
        def _():
            h = s_ref[...] + a_ref[...]
            h_s[...] = h
            q_s[...] = (jnp.dot(h.astype(jnp.bfloat16), wq_ref[...],
                                preferred_element_type=jnp.float32)
                        * scale).astype(jnp.bfloat16)
            acc_s[...] = jnp.zeros_like(acc_s)
            m_s[...] = jnp.full_like(m_s, -jnp.inf)
            l_s[...] = jnp.zeros_like(l_s)

        lo = sched_ref[0, i]
        hi = sched_ref[1, i]

        @pl.when((j >= lo) & (j <= hi))
        def _():
            sc = lax.dot_general(
                q_s[...], k_ref[...], (((1,), (1,)), ((), ())),
                preferred_element_type=jnp.float32)
            yb = yb_ref[0]
            xb = xb_ref[0]
            mask = yb == xb
            sc = jnp.where(mask, sc, NEG_INF)
            m_prev = m_s[...]
            m_new = jnp.maximum(m_prev, jnp.max(sc, axis=1, keepdims=True))
            p = jnp.exp(sc - m_new)
            corr = jnp.exp(m_prev - m_new)
            l_s[...] = l_s[...] * corr + jnp.sum(p, axis=1, keepdims=True)
            m_s[...] = m_new
            acc_s[...] = acc_s[...] * corr + jnp.dot(
                p.astype(jnp.bfloat16), v_ref[...],
                preferred_element_type=jnp.float32)

        @pl.when(j == nx - 1)
        def _():
            o_ref[...] = jnp.maximum(
                h_s[...] + acc_s[...] / l_s[...], 0.0).astype(o_ref.dtype)

    def kv_index(col):
        def f(i, j, sched):
            return (jnp.clip(j, sched[0, i], sched[1, i]), col)
        return f

    def xb_index(i, j, sched):
        return (jnp.clip(j, sched[0, i], sched[1, i]), 0, 0)

    grid_spec = pltpu.PrefetchScalarGridSpec(
        num_scalar_prefetch=1,
        grid=(ny, nx),
        in_specs=[
            pl.BlockSpec((BY, D), lambda i, j, sched: (i, sb_col)),
            pl.BlockSpec((BY, D), lambda i, j, sched: (i, 0)),
            pl.BlockSpec((D, D), lambda i, j, sched: (0, 0)),
            pl.BlockSpec((BX, D), kv_index(kb_col)),
            pl.BlockSpec((BX, D), kv_index(vb_col)),
            pl.BlockSpec((1, BY, 1), lambda i, j, sched: (i, 0, 0)),
            pl.BlockSpec((1, 1, BX), xb_index),
        ],
        out_specs=pl.BlockSpec((BY, D), lambda i, j, sched: (i, 0)),
        scratch_shapes=[
            pltpu.VMEM((BY, D), jnp.float32),
            pltpu.VMEM((BY, D), jnp.bfloat16),
            pltpu.VMEM((BY, D), jnp.float32),
            pltpu.VMEM((BY, 1), jnp.float32),
            pltpu.VMEM((BY, 1), jnp.float32),
        ],
    )
    return pl.pallas_call(
        body,
        grid_spec=grid_spec,
        out_shape=jax.ShapeDtypeStruct((NYr, D), out_dtype),
        compiler_params=pltpu.CompilerParams(
            dimension_semantics=("parallel", "arbitrary")),
    )(sched, s_arr, agg, wq, kv, kv, yb3, xb3)


def _flash_layer2(s_arr, s_col, agg, wq, kv, k_col, v_col,
                  yb3, xb2, sched, D, BY=128, BX=512,
                  out_dtype=jnp.float32):
    NYr = agg.shape[0]
    ny = NYr // BY
    scale = 1.0 / math.sqrt(float(D))
    sb_col = s_col // D
    bf16 = jnp.bfloat16

    def body(sched_ref, s_ref, a_ref, wq_ref, kv_ref, yb_ref, xb_ref, o_ref,
             kbuf, vbuf, acc_s, m_s, l_s, ksem, vsem):
        i = pl.program_id(0)
        lo = sched_ref[0, i]
        hi = sched_ref[1, i]
        h = s_ref[...] + a_ref[...]
        q = (jnp.dot(h.astype(bf16), wq_ref[...],
                     preferred_element_type=jnp.float32) * scale).astype(bf16)
        yb = yb_ref[0]

        def start_copy(jx, slot):
            pltpu.make_async_copy(
                kv_ref.at[pl.ds(jx * BX, BX), pl.ds(k_col, D)],
                kbuf.at[slot], ksem.at[slot]).start()
            pltpu.make_async_copy(
                kv_ref.at[pl.ds(jx * BX, BX), pl.ds(v_col, D)],
                vbuf.at[slot], vsem.at[slot]).start()

        start_copy(lo, 0)
        acc = jnp.zeros((BY, D), jnp.float32)
        acc_s[...] = acc
        m_s[...] = jnp.full((BY, 1), -jnp.inf, jnp.float32)
        l_s[...] = jnp.zeros((BY, 1), jnp.float32)

        def step(j, slot):
            @pl.when(j < hi)
            def _():
                start_copy(j + 1, 1 - slot)

            pltpu.make_async_copy(
                kv_ref.at[pl.ds(j * BX, BX), pl.ds(k_col, D)],
                kbuf.at[slot], ksem.at[slot]).wait()
            pltpu.make_async_copy(
                kv_ref.at[pl.ds(j * BX, BX), pl.ds(v_col, D)],
                vbuf.at[slot], vsem.at[slot]).wait()
            sc = lax.dot_general(
                q, kbuf[slot], (((1,), (1,)), ((), ())),
                preferred_element_type=jnp.float32)
            xb = xb_ref[:, pl.ds(j * BX, BX)]
            sc = jnp.where(yb == xb, sc, NEG_INF)
            m_prev = m_s[...]
            m_new = jnp.maximum(m_prev, jnp.max(sc, axis=1, keepdims=True))
            p = jnp.exp(sc - m_new)
            corr = jnp.exp(m_prev - m_new)
            l_s[...] = l_s[...] * corr + jnp.sum(p, axis=1, keepdims=True)
            m_s[...] = m_new
            acc_s[...] = acc_s[...] * corr + jnp.dot(
                p.astype(bf16), vbuf[slot],
                preferred_element_type=jnp.float32)
            return 1 - slot

        lax.fori_loop(lo, hi + 1, step, 0)
        o_ref[...] = jnp.maximum(
            h + acc_s[...] / l_s[...], 0.0).astype(o_ref.dtype)

    grid_spec = pltpu.PrefetchScalarGridSpec(
        num_scalar_prefetch=1,
        grid=(ny,),
        in_specs=[
            pl.BlockSpec((BY, D), lambda i, sched: (i, sb_col)),
            pl.BlockSpec((BY, D), lambda i, sched: (i, 0)),
            pl.BlockSpec((D, D), lambda i, sched: (0, 0)),
            pl.BlockSpec(memory_space=pl.ANY),
            pl.BlockSpec((1, BY, 1), lambda i, sched: (i, 0, 0)),
            pl.BlockSpec((1, xb2.shape[1]), lambda i, sched: (0, 0)),
        ],
        out_specs=pl.BlockSpec((BY, D), lambda i, sched: (i, 0)),
        scratch_shapes=[
            pltpu.VMEM((2, BX, D), bf16),
            pltpu.VMEM((2, BX, D), bf16),
            pltpu.VMEM((BY, D), jnp.float32),
            pltpu.VMEM((BY, 1), jnp.float32),
            pltpu.VMEM((BY, 1), jnp.float32),
            pltpu.SemaphoreType.DMA((2,)),
            pltpu.SemaphoreType.DMA((2,)),
        ],
    )
    return pl.pallas_call(
        body,
        grid_spec=grid_spec,
        out_shape=jax.ShapeDtypeStruct((NYr, D), out_dtype),
        compiler_params=pltpu.CompilerParams(
            dimension_semantics=("parallel",)),
    )(sched, s_arr, agg, wq, kv, yb3, xb2)


def _block_schedule(y_batch, x_batch, BY, BX, nx):
    ny = y_batch.shape[0] // BY
    b_lo = y_batch[::BY]
    b_hi = y_batch[BY - 1::BY]
    bounds = jnp.searchsorted(x_batch, jnp.arange(NUM_BATCHES + 1),
                              side='left').astype(jnp.int32)
    xs = bounds[b_lo]
    xe = bounds[b_hi + 1]
    lo = jnp.clip(xs // BX, 0, nx - 1)
    hi = jnp.clip(jnp.maximum((xe - 1) // BX, lo), 0, nx - 1)
    return jnp.stack([lo, hi]).astype(jnp.int32)


def kernel(x, x_batch, tgt_y, tgt_edge_index, tgt_edge_type, tgt_y_batch,
           params):
    p = params
    g1, g2, g3 = p['gcn1'], p['gcn2'], p['gcn3']
    src = tgt_edge_index[0]
    dst = tgt_edge_index[1]
    H1 = g1['W_self'].shape[1]
    H2 = g2['W_self'].shape[1]
    H3 = g3['W_self'].shape[1]

    BY, BX = 128, 512
    NXr = x.shape[0]
    nx = NXr // BX
    ny = tgt_y_batch.shape[0] // BY
    sched = _block_schedule(tgt_y_batch, x_batch, BY, BX, nx)
    yb3 = tgt_y_batch.reshape(ny, BY, 1)
    xb2 = x_batch.reshape(1, NXr)

    bf16 = jnp.bfloat16

    kvw = jnp.concatenate(
        [g1['Wk'], g1['Wv'], g2['Wk'], g2['Wv'], g3['Wk'], g3['Wv']], axis=1)
    kv = _matmul(x.astype(bf16), kvw.astype(bf16), out_dtype=bf16, bm=2048)

    ny_nodes = tgt_y.shape[0]
    adj, cnt = _sc_build_adj(src, dst, tgt_edge_type, ny_nodes)
    adjb = adj.astype(bf16)
    cntb = cnt.astype(bf16)

    def _emb_pad(emb):
        return jnp.zeros((16, emb.shape[1]), bf16).at[
            :emb.shape[0]].set(emb.astype(bf16))

    y0 = _sc_gather_rows(p['embed'], tgt_y)
    y0b = y0.astype(bf16)
    s1 = _matmul(y0b, g1['W_self'].astype(bf16))
    ay0 = _matmul(adjb, y0b, out_dtype=bf16)
    agg1 = _mm2(ay0, g1['W_nb'].astype(bf16), cntb, _emb_pad(g1['edge_emb']))
    y1 = _flash_layer2(s1, 0, agg1, g1['Wq'].astype(bf16), kv, 0, H1,
                       yb3, xb2, sched, H1, BY, BX, out_dtype=bf16)

    s2 = _matmul(y1, g2['W_self'].astype(bf16))
    m2 = _matmul(y1, g2['W_nb'].astype(bf16), out_dtype=bf16)
    agg2 = _mm2(adjb, m2, cntb, _emb_pad(g2['edge_emb']))
    y2 = _flash_layer2(s2, 0, agg2, g2['Wq'].astype(bf16), kv,
                       2 * H1, 2 * H1 + H2, yb3, xb2, sched, H2, BY, BX,
                       out_dtype=bf16)

    s3 = _matmul(y2, g3['W_self'].astype(bf16))
    m3 = _matmul(y2, g3['W_nb'].astype(bf16), out_dtype=bf16)
    agg3 = _mm2(adjb, m3, cntb, _emb_pad(g3['edge_emb']))
    y3 = _flash_layer2(s3, 0, agg3, g3['Wq'].astype(bf16), kv,
                       2 * (H1 + H2), 2 * (H1 + H2) + H3,
                       yb3, xb2, sched, H3, BY, BX)

    y3b = y3.astype(bf16)
    y_score = _matmul(y3b, p['Wz'].astype(bf16), bias=p['bz'])

    emb_d = y3.shape[1]
    n_rel = p['Wg'].shape[1]
    wg_pad = jnp.zeros((emb_d, 128), jnp.float32)
    wg_pad = wg_pad.at[:, 0:n_rel].set(p['Wg'][:emb_d])
    wg_pad = wg_pad.at[:, 16:16 + n_rel].set(p['Wg'][emb_d:])
    bg_pad = jnp.zeros((128,), jnp.float32).at[0:n_rel].set(p['bg'])
    ab = _matmul(y3b, wg_pad.astype(bf16), bias=bg_pad, bn=128)
    er = _sc_edge_scores(ab, src, dst)
    y_edge_rel_score = lax.slice_in_dim(er, 0, n_rel, axis=1)

    return (y3, tgt_y_batch, tgt_edge_index, tgt_edge_type, y_score,
            y_edge_rel_score)


# speedup vs baseline: 1.5813x; 1.3055x over previous
import dataclasses
import functools
import math

import jax
import jax.numpy as jnp
from jax import lax
from jax.experimental import pallas as pl
from jax.experimental.pallas import tpu as pltpu
from jax.experimental.pallas import tpu_sc as plsc

N_SC_CORES = 2
N_SUBCORES = 16
N_WORKERS = N_SC_CORES * N_SUBCORES
NUM_BATCHES = 16
NEG_INF = -1e9


def _mm_body(a_ref, b_ref, o_ref):
    o_ref[...] = jax.lax.dot_general(
        a_ref[...].astype(jnp.bfloat16), b_ref[...].astype(jnp.bfloat16),
        (((1,), (0,)), ((), ())),
        preferred_element_type=jnp.float32).astype(o_ref.dtype)


def _mm_bias_body(a_ref, b_ref, bias_ref, o_ref):
    o_ref[...] = (jax.lax.dot_general(
        a_ref[...].astype(jnp.bfloat16), b_ref[...].astype(jnp.bfloat16),
        (((1,), (0,)), ((), ())),
        preferred_element_type=jnp.float32)
        + bias_ref[...]).astype(o_ref.dtype)


def _mm2_body(a1_ref, b1_ref, a2_ref, b2_ref, o_ref):
    o_ref[...] = (jax.lax.dot_general(
        a1_ref[...], b1_ref[...], (((1,), (0,)), ((), ())),
        preferred_element_type=jnp.float32) + jax.lax.dot_general(
        a2_ref[...], b2_ref[...], (((1,), (0,)), ((), ())),
        preferred_element_type=jnp.float32)).astype(o_ref.dtype)


def _mm2(a1, b1, a2, b2, bm=512, bn=512, out_dtype=jnp.float32):
    M, K1 = a1.shape
    _, N = b1.shape
    K2 = a2.shape[1]
    bm = min(bm, M)
    bn = min(bn, N)
    grid = (M // bm, N // bn)
    return pl.pallas_call(
        _mm2_body,
        grid=grid,
        in_specs=[
            pl.BlockSpec((bm, K1), lambda i, j: (i, 0)),
            pl.BlockSpec((K1, bn), lambda i, j: (0, j)),
            pl.BlockSpec((bm, K2), lambda i, j: (i, 0)),
            pl.BlockSpec((K2, bn), lambda i, j: (0, j)),
        ],
        out_specs=pl.BlockSpec((bm, bn), lambda i, j: (i, j)),
        out_shape=jax.ShapeDtypeStruct((M, N), out_dtype),
        compiler_params=pltpu.CompilerParams(
            dimension_semantics=("parallel", "parallel")),
    )(a1, b1, a2, b2)


def _matmul(a, b, bias=None, bm=512, bn=512, out_dtype=jnp.float32):
    M, K = a.shape
    _, N = b.shape
    bm = min(bm, M)
    bn = min(bn, N)
    grid = (M // bm, N // bn)
    in_specs = [
        pl.BlockSpec((bm, K), lambda i, j: (i, 0)),
        pl.BlockSpec((K, bn), lambda i, j: (0, j)),
    ]
    args = [a, b]
    body = _mm_body
    if bias is not None:
        in_specs.append(pl.BlockSpec((1, bn), lambda i, j: (0, j)))
        args.append(bias.reshape(1, N))
        body = _mm_bias_body
    return pl.pallas_call(
        body,
        grid=grid,
        in_specs=in_specs,
        out_specs=pl.BlockSpec((bm, bn), lambda i, j: (i, j)),
        out_shape=jax.ShapeDtypeStruct((M, N), out_dtype),
        compiler_params=pltpu.CompilerParams(
            dimension_semantics=("parallel", "parallel")),
    )(*args)


def _sc_gather_rows(table, idx):
    n = idx.shape[0]
    _, D = table.shape
    per_w = n // N_WORKERS
    chunk = min(per_w, 64)
    nchunks = per_w // chunk
    mesh = plsc.VectorSubcoreMesh(core_axis_name="c", subcore_axis_name="s")

    @functools.partial(
        pl.kernel, mesh=mesh,
        out_type=jax.ShapeDtypeStruct((n, D), jnp.float32),
        scratch_types=[
            pltpu.VMEM((chunk,), jnp.int32),
            pltpu.VMEM((chunk, D), jnp.float32),
            pltpu.SemaphoreType.DMA,
        ])
    def k(table_hbm, idx_hbm, out_hbm, idx_v, rows_v, sem):
        wid = lax.axis_index("s") * N_SC_CORES + lax.axis_index("c")
        base = wid * per_w

        @pl.loop(0, nchunks)
        def _(c):
            b = base + c * chunk
            pltpu.sync_copy(idx_hbm.at[pl.ds(b, chunk)], idx_v)
            pltpu.async_copy(table_hbm.at[idx_v], rows_v, sem).wait()
            pltpu.sync_copy(rows_v, out_hbm.at[pl.ds(b, chunk)])

    return k(table, idx)


def _sc_build_adj(src, dst, etype, ny):
    E = src.shape[0]
    per_rows = ny // N_WORKERS
    half = per_rows // 2
    mesh = plsc.VectorSubcoreMesh(core_axis_name="c", subcore_axis_name="s")

    cp = pltpu.CompilerParams()
    if "needs_layout_passes" in pltpu.CompilerParams.__dataclass_fields__:
        cp = dataclasses.replace(cp, needs_layout_passes=False)

    @functools.partial(
        pl.kernel, mesh=mesh, compiler_params=cp,
        out_type=(jax.ShapeDtypeStruct((ny, ny), jnp.float32),
                  jax.ShapeDtypeStruct((ny, 16), jnp.float32)),
        scratch_types=[
            pltpu.VMEM((E,), jnp.int32),
            pltpu.VMEM((E,), jnp.int32),
            pltpu.VMEM((E,), jnp.int32),
            pltpu.VMEM((E + 16,), jnp.int32),
            pltpu.VMEM((E + 16,), jnp.int32),
            pltpu.VMEM((E + 16,), jnp.int32),
            pltpu.VMEM((half, ny), jnp.float32),
            pltpu.VMEM((per_rows, 16), jnp.float32),
        ])
    def k(src_hbm, dst_hbm, et_hbm, adj_hbm, cnt_hbm,
          src_v, dst_v, et_v, cs_v, cd_v, ct_v, acc_v, c_v):
        cid = lax.axis_index("c")
        sid = lax.axis_index("s")
        wid = sid * N_SC_CORES + cid
        row0 = wid * per_rows
        pltpu.sync_copy(src_hbm, src_v)
        pltpu.sync_copy(dst_hbm, dst_v)
        pltpu.sync_copy(et_hbm, et_v)

        @pl.loop(0, per_rows)
        def _(r):
            c_v.at[r, pl.ds(0, 16)][...] = jnp.zeros((16,), jnp.float32)

        lanes = lax.iota(jnp.int32, 16)
        ones = jnp.full((16,), 1.0, jnp.float32)

        def compress(g, off):
            b = g * 16
            d_v = dst_v[pl.ds(b, 16)]
            m = (d_v >= row0) & (d_v < row0 + per_rows)
            plsc.store_compressed(cd_v.at[pl.ds(off, 16)], d_v, mask=m)
            plsc.store_compressed(cs_v.at[pl.ds(off, 16)],
                                  src_v[pl.ds(b, 16)], mask=m)
            plsc.store_compressed(ct_v.at[pl.ds(off, 16)],
                                  et_v[pl.ds(b, 16)], mask=m)
            return off + jnp.sum(m.astype(jnp.int32), axis=0)

        nmine = lax.fori_loop(0, E // 16, compress, jnp.int32(0))
        ng = (nmine + 15) // 16

        def cnt_pass(g, _):
            b = g * 16
            valid = lanes < (nmine - b)
            d_v = cd_v[pl.ds(b, 16)]
            t_v = ct_v[pl.ds(b, 16)]
            rc_v = jnp.clip(d_v - row0, 0, per_rows - 1)
            for l in range(16):
                plsc.addupdate_scatter(
                    c_v, [rc_v, t_v], ones, mask=valid & (lanes == l))
            return 0

        lax.fori_loop(0, ng, cnt_pass, 0)

        for p in range(2):
            plo = row0 + p * half

            @pl.loop(0, half)
            def _(r):
                @pl.loop(0, ny, step=16)
                def _(cc):
                    acc_v.at[r, pl.ds(cc, 16)][...] = jnp.zeros(
                        (16,), jnp.float32)

            def adj_pass(g, _):
                b = g * 16
                valid = lanes < (nmine - b)
                d_v = cd_v[pl.ds(b, 16)]
                s_v = cs_v[pl.ds(b, 16)]
                in_a = valid & (d_v >= plo) & (d_v < plo + half)
                r_v = jnp.clip(d_v - plo, 0, half - 1)
                for l in range(16):
                    plsc.addupdate_scatter(
                        acc_v, [r_v, s_v], ones, mask=in_a & (lanes == l))
                return 0

            lax.fori_loop(0, ng, adj_pass, 0)
            pltpu.sync_copy(acc_v, adj_hbm.at[pl.ds(plo, half)])

        pltpu.sync_copy(c_v, cnt_hbm.at[pl.ds(row0, per_rows)])

    return k(src, dst, etype)


def _sc_edge_scores(ab, src, dst):
    E = src.shape[0]
    W = ab.shape[1]
    per_w = E // N_WORKERS
    chunk = min(per_w, 64)
    nchunks = per_w // chunk
    mesh = plsc.VectorSubcoreMesh(core_axis_name="c", subcore_axis_name="s")

    cp = pltpu.CompilerParams()
    if "needs_layout_passes" in pltpu.CompilerParams.__dataclass_fields__:
        cp = dataclasses.replace(cp, needs_layout_passes=False)

    @functools.partial(
        pl.kernel, mesh=mesh, compiler_params=cp,
        out_type=jax.ShapeDtypeStruct((E, 16), jnp.float32),
        scratch_types=[
            pltpu.VMEM((chunk,), jnp.int32),
            pltpu.VMEM((chunk,), jnp.int32),
            pltpu.VMEM((chunk, W), jnp.float32),
            pltpu.VMEM((chunk, W), jnp.float32),
            pltpu.VMEM((chunk, 16), jnp.float32),
            pltpu.SemaphoreType.DMA,
            pltpu.SemaphoreType.DMA,
        ])
    def k(ab_hbm, src_hbm, dst_hbm, out_hbm,
          src_v, dst_v, arow_v, brow_v, o_v, sem1, sem2):
        wid = lax.axis_index("s") * N_SC_CORES + lax.axis_index("c")
        base = wid * per_w
        lanes = lax.iota(jnp.int32, 16)

        @pl.loop(0, nchunks)
        def _(c):
            b = base + c * chunk
            pltpu.sync_copy(src_hbm.at[pl.ds(b, chunk)], src_v)
            pltpu.sync_copy(dst_hbm.at[pl.ds(b, chunk)], dst_v)
            cp1 = pltpu.async_copy(ab_hbm.at[src_v], arow_v, sem1)
            cp2 = pltpu.async_copy(ab_hbm.at[dst_v], brow_v, sem2)
            cp1.wait()
            cp2.wait()

            @pl.loop(0, chunk)
            def _(i):
                row = jnp.full((16,), 0, jnp.int32) + i
                a = plsc.load_gather(arow_v, [row, lanes])
                bb = plsc.load_gather(brow_v, [row, lanes + 16])
                plsc.store_scatter(o_v, [row, lanes], a + bb)

            pltpu.sync_copy(o_v, out_hbm.at[pl.ds(b, chunk)])

    return k(ab, src, dst)


def _flash_layer(s_arr, s_col, agg, wq, kv, k_col, v_col,
                 yb3, xb3, sched, D, BY=128, BX=512,
                 out_dtype=jnp.float32):
    NYr = agg.shape[0]
    NXr = kv.shape[0]
    ny = NYr // BY
    nx = NXr // BX
    scale = 1.0 / math.sqrt(float(D))
    kb_col = k_col // D
    vb_col = v_col // D
    sb_col = s_col // D

    def body(sched_ref, s_ref, a_ref, wq_ref, k_ref, v_ref,
             yb_ref, xb_ref, o_ref, h_s, q_s, acc_s, m_s, l_s):
        i = pl.program_id(0)
        j = pl.program_id(1)

        @pl.when(j == 0)
        def _():
            h = s_ref[...] + a_ref[...]
            h_s[...] = h
            q_s[...] = (jnp.dot(h.astype(jnp.bfloat16), wq_ref[...],
                                preferred_element_type=jnp.float32)
                        * scale).astype(jnp.bfloat16)
            acc_s[...] = jnp.zeros_like(acc_s)
            m_s[...] = jnp.full_like(m_s, -jnp.inf)
            l_s[...] = jnp.zeros_like(l_s)

        lo = sched_ref[0, i]
        hi = sched_ref[1, i]

        @pl.when((j >= lo) & (j <= hi))
        def _():
            sc = lax.dot_general(
                q_s[...], k_ref[...], (((1,), (1,)), ((), ())),
                preferred_element_type=jnp.float32)
            yb = yb_ref[0]
            xb = xb_ref[0]
            mask = yb == xb
            sc = jnp.where(mask, sc, NEG_INF)
            m_prev = m_s[...]
            m_new = jnp.maximum(m_prev, jnp.max(sc, axis=1, keepdims=True))
            p = jnp.exp(sc - m_new)
            corr = jnp.exp(m_prev - m_new)
            l_s[...] = l_s[...] * corr + jnp.sum(p, axis=1, keepdims=True)
            m_s[...] = m_new
            acc_s[...] = acc_s[...] * corr + jnp.dot(
                p.astype(jnp.bfloat16), v_ref[...],
                preferred_element_type=jnp.float32)

        @pl.when(j == nx - 1)
        def _():
            o_ref[...] = jnp.maximum(
                h_s[...] + acc_s[...] / l_s[...], 0.0).astype(o_ref.dtype)

    def kv_index(col):
        def f(i, j, sched):
            return (jnp.clip(j, sched[0, i], sched[1, i]), col)
        return f

    def xb_index(i, j, sched):
        return (jnp.clip(j, sched[0, i], sched[1, i]), 0, 0)

    grid_spec = pltpu.PrefetchScalarGridSpec(
        num_scalar_prefetch=1,
        grid=(ny, nx),
        in_specs=[
            pl.BlockSpec((BY, D), lambda i, j, sched: (i, sb_col)),
            pl.BlockSpec((BY, D), lambda i, j, sched: (i, 0)),
            pl.BlockSpec((D, D), lambda i, j, sched: (0, 0)),
            pl.BlockSpec((BX, D), kv_index(kb_col)),
            pl.BlockSpec((BX, D), kv_index(vb_col)),
            pl.BlockSpec((1, BY, 1), lambda i, j, sched: (i, 0, 0)),
            pl.BlockSpec((1, 1, BX), xb_index),
        ],
        out_specs=pl.BlockSpec((BY, D), lambda i, j, sched: (i, 0)),
        scratch_shapes=[
            pltpu.VMEM((BY, D), jnp.float32),
            pltpu.VMEM((BY, D), jnp.bfloat16),
            pltpu.VMEM((BY, D), jnp.float32),
            pltpu.VMEM((BY, 1), jnp.float32),
            pltpu.VMEM((BY, 1), jnp.float32),
        ],
    )
    return pl.pallas_call(
        body,
        grid_spec=grid_spec,
        out_shape=jax.ShapeDtypeStruct((NYr, D), out_dtype),
        compiler_params=pltpu.CompilerParams(
            dimension_semantics=("parallel", "arbitrary")),
    )(sched, s_arr, agg, wq, kv, kv, yb3, xb3)


def _flash_layer2(s_arr, s_col, agg, wq, kv, k_col, v_col,
                  yb3, xb2, sched, D, BY=128, BX=512,
                  out_dtype=jnp.float32):
    NYr = agg.shape[0]
    ny = NYr // BY
    scale = 1.0 / math.sqrt(float(D))
    sb_col = s_col // D
    bf16 = jnp.bfloat16

    def body(sched_ref, s_ref, a_ref, wq_ref, kv_ref, yb_ref, xb_ref, o_ref,
             kbuf, vbuf, acc_s, m_s, l_s, ksem, vsem):
        i = pl.program_id(0)
        lo = sched_ref[0, i]
        hi = sched_ref[1, i]
        h = s_ref[...] + a_ref[...]
        q = (jnp.dot(h.astype(bf16), wq_ref[...],
                     preferred_element_type=jnp.float32) * scale).astype(bf16)
        yb = yb_ref[0]

        def start_copy(jx, slot):
            pltpu.make_async_copy(
                kv_ref.at[pl.ds(jx * BX, BX), pl.ds(k_col, D)],
                kbuf.at[slot], ksem.at[slot]).start()
            pltpu.make_async_copy(
                kv_ref.at[pl.ds(jx * BX, BX), pl.ds(v_col, D)],
                vbuf.at[slot], vsem.at[slot]).start()

        start_copy(lo, 0)
        acc = jnp.zeros((BY, D), jnp.float32)
        acc_s[...] = acc
        m_s[...] = jnp.full((BY, 1), -jnp.inf, jnp.float32)
        l_s[...] = jnp.zeros((BY, 1), jnp.float32)

        def step(j, slot):
            @pl.when(j < hi)
            def _():
                start_copy(j + 1, 1 - slot)

            pltpu.make_async_copy(
                kv_ref.at[pl.ds(j * BX, BX), pl.ds(k_col, D)],
                kbuf.at[slot], ksem.at[slot]).wait()
            pltpu.make_async_copy(
                kv_ref.at[pl.ds(j * BX, BX), pl.ds(v_col, D)],
                vbuf.at[slot], vsem.at[slot]).wait()
            sc = lax.dot_general(
                q, kbuf[slot], (((1,), (1,)), ((), ())),
                preferred_element_type=jnp.float32)
            xb = xb_ref[:, pl.ds(j * BX, BX)]
            sc = jnp.where(yb == xb, sc, NEG_INF)
            m_prev = m_s[...]
            m_new = jnp.maximum(m_prev, jnp.max(sc, axis=1, keepdims=True))
            p = jnp.exp(sc - m_new)
            corr = jnp.exp(m_prev - m_new)
            l_s[...] = l_s[...] * corr + jnp.sum(p, axis=1, keepdims=True)
            m_s[...] = m_new
            acc_s[...] = acc_s[...] * corr + jnp.dot(
                p.astype(bf16), vbuf[slot],
                preferred_element_type=jnp.float32)
            return 1 - slot

        lax.fori_loop(lo, hi + 1, step, 0)
        o_ref[...] = jnp.maximum(
            h + acc_s[...] / l_s[...], 0.0).astype(o_ref.dtype)

    grid_spec = pltpu.PrefetchScalarGridSpec(
        num_scalar_prefetch=1,
        grid=(ny,),
        in_specs=[
            pl.BlockSpec((BY, D), lambda i, sched: (i, sb_col)),
            pl.BlockSpec((BY, D), lambda i, sched: (i, 0)),
            pl.BlockSpec((D, D), lambda i, sched: (0, 0)),
            pl.BlockSpec(memory_space=pl.ANY),
            pl.BlockSpec((1, BY, 1), lambda i, sched: (i, 0, 0)),
            pl.BlockSpec((1, xb2.shape[1]), lambda i, sched: (0, 0)),
        ],
        out_specs=pl.BlockSpec((BY, D), lambda i, sched: (i, 0)),
        scratch_shapes=[
            pltpu.VMEM((2, BX, D), bf16),
            pltpu.VMEM((2, BX, D), bf16),
            pltpu.VMEM((BY, D), jnp.float32),
            pltpu.VMEM((BY, 1), jnp.float32),
            pltpu.VMEM((BY, 1), jnp.float32),
            pltpu.SemaphoreType.DMA((2,)),
            pltpu.SemaphoreType.DMA((2,)),
        ],
    )
    return pl.pallas_call(
        body,
        grid_spec=grid_spec,
        out_shape=jax.ShapeDtypeStruct((NYr, D), out_dtype),
        compiler_params=pltpu.CompilerParams(
            dimension_semantics=("parallel",)),
    )(sched, s_arr, agg, wq, kv, yb3, xb2)


def _flash_layer3(y_prev, w1, A, B, cnt, embp, wq, kv, k_col, v_col,
                  yb3, xb2, sched, D, BY=256, BX=512,
                  out_dtype=jnp.float32):
    NYr = A.shape[0]
    KA = A.shape[1]
    in_d = y_prev.shape[1]
    ny = NYr // BY
    scale = 1.0 / math.sqrt(float(D))
    bf16 = jnp.bfloat16

    def body(sched_ref, y_ref, w1_ref, a_ref, b_ref, cnt_ref, embp_ref,
             wq_ref, kv_ref, yb_ref, xb_ref, o_ref,
             kbuf, vbuf, acc_s, m_s, l_s, ksem, vsem):
        i = pl.program_id(0)
        lo = sched_ref[0, i]
        hi = sched_ref[1, i]

        def start_copy(jx, slot):
            pltpu.make_async_copy(
                kv_ref.at[pl.ds(jx * BX, BX), pl.ds(k_col, D)],
                kbuf.at[slot], ksem.at[slot]).start()
            pltpu.make_async_copy(
                kv_ref.at[pl.ds(jx * BX, BX), pl.ds(v_col, D)],
                vbuf.at[slot], vsem.at[slot]).start()

        start_copy(lo, 0)
        h = (jnp.dot(y_ref[...].astype(bf16), w1_ref[...],
                     preferred_element_type=jnp.float32)
             + jnp.dot(a_ref[...].astype(bf16), b_ref[...],
                       preferred_element_type=jnp.float32)
             + jnp.dot(cnt_ref[...].astype(bf16), embp_ref[...],
                       preferred_element_type=jnp.float32))
        q = (jnp.dot(h.astype(bf16), wq_ref[...],
                     preferred_element_type=jnp.float32) * scale).astype(bf16)
        yb = yb_ref[0]

        acc_s[...] = jnp.zeros((BY, D), jnp.float32)
        m_s[...] = jnp.full((BY, 1), -jnp.inf, jnp.float32)
        l_s[...] = jnp.zeros((BY, 1), jnp.float32)

        def step(j, slot):
            @pl.when(j < hi)
            def _():
                start_copy(j + 1, 1 - slot)

            pltpu.make_async_copy(
                kv_ref.at[pl.ds(j * BX, BX), pl.ds(k_col, D)],
                kbuf.at[slot], ksem.at[slot]).wait()
            pltpu.make_async_copy(
                kv_ref.at[pl.ds(j * BX, BX), pl.ds(v_col, D)],
                vbuf.at[slot], vsem.at[slot]).wait()
            sc = lax.dot_general(
                q, kbuf[slot], (((1,), (1,)), ((), ())),
                preferred_element_type=jnp.float32)
            xb = xb_ref[:, pl.ds(j * BX, BX)]
            sc = jnp.where(yb == xb, sc, NEG_INF)
            m_prev = m_s[...]
            m_new = jnp.maximum(m_prev, jnp.max(sc, axis=1, keepdims=True))
            p = jnp.exp(sc - m_new)
            corr = jnp.exp(m_prev - m_new)
            l_s[...] = l_s[...] * corr + jnp.sum(p, axis=1, keepdims=True)
            m_s[...] = m_new
            acc_s[...] = acc_s[...] * corr + jnp.dot(
                p.astype(bf16), vbuf[slot],
                preferred_element_type=jnp.float32)
            return 1 - slot

        lax.fori_loop(lo, hi + 1, step, 0)
        o_ref[...] = jnp.maximum(
            h + acc_s[...] / l_s[...], 0.0).astype(o_ref.dtype)

    grid_spec = pltpu.PrefetchScalarGridSpec(
        num_scalar_prefetch=1,
        grid=(ny,),
        in_specs=[
            pl.BlockSpec((BY, in_d), lambda i, sched: (i, 0)),
            pl.BlockSpec((in_d, D), lambda i, sched: (0, 0)),
            pl.BlockSpec((BY, KA), lambda i, sched: (i, 0)),
            pl.BlockSpec((KA, D), lambda i, sched: (0, 0)),
            pl.BlockSpec((BY, 16), lambda i, sched: (i, 0)),
            pl.BlockSpec((16, D), lambda i, sched: (0, 0)),
            pl.BlockSpec((D, D), lambda i, sched: (0, 0)),
            pl.BlockSpec(memory_space=pl.ANY),
            pl.BlockSpec((1, BY, 1), lambda i, sched: (i, 0, 0)),
            pl.BlockSpec((1, xb2.shape[1]), lambda i, sched: (0, 0)),
        ],
        out_specs=pl.BlockSpec((BY, D), lambda i, sched: (i, 0)),
        scratch_shapes=[
            pltpu.VMEM((2, BX, D), bf16),
            pltpu.VMEM((2, BX, D), bf16),
            pltpu.VMEM((BY, D), jnp.float32),
            pltpu.VMEM((BY, 1), jnp.float32),
            pltpu.VMEM((BY, 1), jnp.float32),
            pltpu.SemaphoreType.DMA((2,)),
            pltpu.SemaphoreType.DMA((2,)),
        ],
    )
    return pl.pallas_call(
        body,
        grid_spec=grid_spec,
        out_shape=jax.ShapeDtypeStruct((NYr, D), out_dtype),
        compiler_params=pltpu.CompilerParams(
            dimension_semantics=("parallel",)),
    )(sched, y_prev, w1, A, B, cnt, embp, wq, kv, yb3, xb2)


def _block_schedule(y_batch, x_batch, BY, BX, nx):
    ny = y_batch.shape[0] // BY
    b_lo = y_batch[::BY]
    b_hi = y_batch[BY - 1::BY]
    bounds = jnp.searchsorted(x_batch, jnp.arange(NUM_BATCHES + 1),
                              side='left').astype(jnp.int32)
    xs = bounds[b_lo]
    xe = bounds[b_hi + 1]
    lo = jnp.clip(xs // BX, 0, nx - 1)
    hi = jnp.clip(jnp.maximum((xe - 1) // BX, lo), 0, nx - 1)
    return jnp.stack([lo, hi]).astype(jnp.int32)


def kernel(x, x_batch, tgt_y, tgt_edge_index, tgt_edge_type, tgt_y_batch,
           params):
    p = params
    g1, g2, g3 = p['gcn1'], p['gcn2'], p['gcn3']
    src = tgt_edge_index[0]
    dst = tgt_edge_index[1]
    H1 = g1['W_self'].shape[1]
    H2 = g2['W_self'].shape[1]
    H3 = g3['W_self'].shape[1]

    BY, BX = 256, 512
    NXr = x.shape[0]
    nx = NXr // BX
    ny = tgt_y_batch.shape[0] // BY
    sched = _block_schedule(tgt_y_batch, x_batch, BY, BX, nx)
    yb3 = tgt_y_batch.reshape(ny, BY, 1)
    xb2 = x_batch.reshape(1, NXr)

    bf16 = jnp.bfloat16

    kvw = jnp.concatenate(
        [g1['Wk'], g1['Wv'], g2['Wk'], g2['Wv'], g3['Wk'], g3['Wv']], axis=1)
    kv = _matmul(x, kvw, out_dtype=bf16, bm=2048)

    ny_nodes = tgt_y.shape[0]
    adj, cnt = _sc_build_adj(src, dst, tgt_edge_type, ny_nodes)

    def _emb_pad(emb):
        return jnp.zeros((16, emb.shape[1]), bf16).at[
            :emb.shape[0]].set(emb.astype(bf16))

    y0 = _sc_gather_rows(p['embed'], tgt_y)
    ay0 = _matmul(adj, y0, out_dtype=bf16)
    y1 = _flash_layer3(y0, g1['W_self'].astype(bf16), ay0,
                       g1['W_nb'].astype(bf16), cnt, _emb_pad(g1['edge_emb']),
                       g1['Wq'].astype(bf16), kv, 0, H1,
                       yb3, xb2, sched, H1, BY, BX, out_dtype=bf16)

    m2 = _matmul(y1, g2['W_nb'], out_dtype=bf16)
    y2 = _flash_layer3(y1, g2['W_self'].astype(bf16), adj, m2, cnt,
                       _emb_pad(g2['edge_emb']), g2['Wq'].astype(bf16), kv,
                       2 * H1, 2 * H1 + H2, yb3, xb2, sched, H2, BY, BX,
                       out_dtype=bf16)

    m3 = _matmul(y2, g3['W_nb'], out_dtype=bf16)
    y3 = _flash_layer3(y2, g3['W_self'].astype(bf16), adj, m3, cnt,
                       _emb_pad(g3['edge_emb']), g3['Wq'].astype(bf16), kv,
                       2 * (H1 + H2), 2 * (H1 + H2) + H3,
                       yb3, xb2, sched, H3, BY, BX)

    emb_d = y3.shape[1]
    n_rel = p['Wg'].shape[1]
    vocab = p['Wz'].shape[1]
    wg_pad = jnp.zeros((emb_d, 128), jnp.float32)
    wg_pad = wg_pad.at[:, 0:n_rel].set(p['Wg'][:emb_d])
    wg_pad = wg_pad.at[:, 16:16 + n_rel].set(p['Wg'][emb_d:])
    bg_pad = jnp.zeros((128,), jnp.float32).at[0:n_rel].set(p['bg'])
    head_w = jnp.concatenate([p['Wz'], wg_pad], axis=1)
    head_b = jnp.concatenate([p['bz'], bg_pad])
    head = _matmul(y3, head_w, bias=head_b, bn=vocab + 128)
    y_score = lax.slice_in_dim(head, 0, vocab, axis=1)
    ab = lax.slice_in_dim(head, vocab, vocab + 128, axis=1)
    er = _sc_edge_scores(ab, src, dst)
    y_edge_rel_score = lax.slice_in_dim(er, 0, n_rel, axis=1)

    return (y3, tgt_y_batch, tgt_edge_index, tgt_edge_type, y_score,
            y_edge_rel_score)


# speedup vs baseline: 1.7185x; 1.0868x over previous
import dataclasses
import functools
import math

import jax
import jax.numpy as jnp
from jax import lax
from jax.experimental import pallas as pl
from jax.experimental.pallas import tpu as pltpu
from jax.experimental.pallas import tpu_sc as plsc

N_SC_CORES = 2
N_SUBCORES = 16
N_WORKERS = N_SC_CORES * N_SUBCORES
NUM_BATCHES = 16
NEG_INF = -1e9


def _mm_body(a_ref, b_ref, o_ref):
    o_ref[...] = jax.lax.dot_general(
        a_ref[...].astype(jnp.bfloat16), b_ref[...].astype(jnp.bfloat16),
        (((1,), (0,)), ((), ())),
        preferred_element_type=jnp.float32).astype(o_ref.dtype)


def _mm_bias_body(a_ref, b_ref, bias_ref, o_ref):
    o_ref[...] = (jax.lax.dot_general(
        a_ref[...].astype(jnp.bfloat16), b_ref[...].astype(jnp.bfloat16),
        (((1,), (0,)), ((), ())),
        preferred_element_type=jnp.float32)
        + bias_ref[...]).astype(o_ref.dtype)


def _mm2_body(a1_ref, b1_ref, a2_ref, b2_ref, o_ref):
    o_ref[...] = (jax.lax.dot_general(
        a1_ref[...], b1_ref[...], (((1,), (0,)), ((), ())),
        preferred_element_type=jnp.float32) + jax.lax.dot_general(
        a2_ref[...], b2_ref[...], (((1,), (0,)), ((), ())),
        preferred_element_type=jnp.float32)).astype(o_ref.dtype)


def _mm2(a1, b1, a2, b2, bm=512, bn=512, out_dtype=jnp.float32):
    M, K1 = a1.shape
    _, N = b1.shape
    K2 = a2.shape[1]
    bm = min(bm, M)
    bn = min(bn, N)
    grid = (M // bm, N // bn)
    return pl.pallas_call(
        _mm2_body,
        grid=grid,
        in_specs=[
            pl.BlockSpec((bm, K1), lambda i, j: (i, 0)),
            pl.BlockSpec((K1, bn), lambda i, j: (0, j)),
            pl.BlockSpec((bm, K2), lambda i, j: (i, 0)),
            pl.BlockSpec((K2, bn), lambda i, j: (0, j)),
        ],
        out_specs=pl.BlockSpec((bm, bn), lambda i, j: (i, j)),
        out_shape=jax.ShapeDtypeStruct((M, N), out_dtype),
        compiler_params=pltpu.CompilerParams(
            dimension_semantics=("parallel", "parallel")),
    )(a1, b1, a2, b2)


def _matmul(a, b, bias=None, bm=512, bn=512, out_dtype=jnp.float32):
    M, K = a.shape
    _, N = b.shape
    bm = min(bm, M)
    bn = min(bn, N)
    grid = (M // bm, N // bn)
    in_specs = [
        pl.BlockSpec((bm, K), lambda i, j: (i, 0)),
        pl.BlockSpec((K, bn), lambda i, j: (0, j)),
    ]
    args = [a, b]
    body = _mm_body
    if bias is not None:
        in_specs.append(pl.BlockSpec((1, bn), lambda i, j: (0, j)))
        args.append(bias.reshape(1, N))
        body = _mm_bias_body
    return pl.pallas_call(
        body,
        grid=grid,
        in_specs=in_specs,
        out_specs=pl.BlockSpec((bm, bn), lambda i, j: (i, j)),
        out_shape=jax.ShapeDtypeStruct((M, N), out_dtype),
        compiler_params=pltpu.CompilerParams(
            dimension_semantics=("parallel", "parallel")),
    )(*args)


def _sc_gather_rows(table, idx):
    n = idx.shape[0]
    _, D = table.shape
    per_w = n // N_WORKERS
    chunk = min(per_w, 64)
    nchunks = per_w // chunk
    mesh = plsc.VectorSubcoreMesh(core_axis_name="c", subcore_axis_name="s")

    @functools.partial(
        pl.kernel, mesh=mesh,
        out_type=jax.ShapeDtypeStruct((n, D), jnp.float32),
        scratch_types=[
            pltpu.VMEM((chunk,), jnp.int32),
            pltpu.VMEM((chunk, D), jnp.float32),
            pltpu.SemaphoreType.DMA,
        ])
    def k(table_hbm, idx_hbm, out_hbm, idx_v, rows_v, sem):
        wid = lax.axis_index("s") * N_SC_CORES + lax.axis_index("c")
        base = wid * per_w

        @pl.loop(0, nchunks)
        def _(c):
            b = base + c * chunk
            pltpu.sync_copy(idx_hbm.at[pl.ds(b, chunk)], idx_v)
            pltpu.async_copy(table_hbm.at[idx_v], rows_v, sem).wait()
            pltpu.sync_copy(rows_v, out_hbm.at[pl.ds(b, chunk)])

    return k(table, idx)


def _sc_build_adj(src, dst, etype, ny):
    E = src.shape[0]
    per_rows = ny // N_WORKERS
    half = per_rows // 2
    mesh = plsc.VectorSubcoreMesh(core_axis_name="c", subcore_axis_name="s")

    cp = pltpu.CompilerParams()
    if "needs_layout_passes" in pltpu.CompilerParams.__dataclass_fields__:
        cp = dataclasses.replace(cp, needs_layout_passes=False)

    @functools.partial(
        pl.kernel, mesh=mesh, compiler_params=cp,
        out_type=(jax.ShapeDtypeStruct((ny, ny), jnp.float32),
                  jax.ShapeDtypeStruct((ny, 16), jnp.float32)),
        scratch_types=[
            pltpu.VMEM((E,), jnp.int32),
            pltpu.VMEM((E,), jnp.int32),
            pltpu.VMEM((E,), jnp.int32),
            pltpu.VMEM((E + 16,), jnp.int32),
            pltpu.VMEM((E + 16,), jnp.int32),
            pltpu.VMEM((E + 16,), jnp.int32),
            pltpu.VMEM((half, ny), jnp.float32),
            pltpu.VMEM((per_rows, 16), jnp.float32),
        ])
    def k(src_hbm, dst_hbm, et_hbm, adj_hbm, cnt_hbm,
          src_v, dst_v, et_v, cs_v, cd_v, ct_v, acc_v, c_v):
        cid = lax.axis_index("c")
        sid = lax.axis_index("s")
        wid = sid * N_SC_CORES + cid
        row0 = wid * per_rows
        pltpu.sync_copy(src_hbm, src_v)
        pltpu.sync_copy(dst_hbm, dst_v)
        pltpu.sync_copy(et_hbm, et_v)

        @pl.loop(0, per_rows)
        def _(r):
            c_v.at[r, pl.ds(0, 16)][...] = jnp.zeros((16,), jnp.float32)

        lanes = lax.iota(jnp.int32, 16)
        ones = jnp.full((16,), 1.0, jnp.float32)

        def compress(g, off):
            b = g * 16
            d_v = dst_v[pl.ds(b, 16)]
            m = (d_v >= row0) & (d_v < row0 + per_rows)
            plsc.store_compressed(cd_v.at[pl.ds(off, 16)], d_v, mask=m)
            plsc.store_compressed(cs_v.at[pl.ds(off, 16)],
                                  src_v[pl.ds(b, 16)], mask=m)
            plsc.store_compressed(ct_v.at[pl.ds(off, 16)],
                                  et_v[pl.ds(b, 16)], mask=m)
            return off + jnp.sum(m.astype(jnp.int32), axis=0)

        nmine = lax.fori_loop(0, E // 16, compress, jnp.int32(0))
        ng = (nmine + 15) // 16

        def cnt_pass(g, _):
            b = g * 16
            valid = lanes < (nmine - b)
            d_v = cd_v[pl.ds(b, 16)]
            t_v = ct_v[pl.ds(b, 16)]
            rc_v = jnp.clip(d_v - row0, 0, per_rows - 1)
            for l in range(16):
                plsc.addupdate_scatter(
                    c_v, [rc_v, t_v], ones, mask=valid & (lanes == l))
            return 0

        lax.fori_loop(0, ng, cnt_pass, 0)

        for p in range(2):
            plo = row0 + p * half

            @pl.loop(0, half)
            def _(r):
                @pl.loop(0, ny, step=16)
                def _(cc):
                    acc_v.at[r, pl.ds(cc, 16)][...] = jnp.zeros(
                        (16,), jnp.float32)

            def adj_pass(g, _):
                b = g * 16
                valid = lanes < (nmine - b)
                d_v = cd_v[pl.ds(b, 16)]
                s_v = cs_v[pl.ds(b, 16)]
                in_a = valid & (d_v >= plo) & (d_v < plo + half)
                r_v = jnp.clip(d_v - plo, 0, half - 1)
                for l in range(16):
                    plsc.addupdate_scatter(
                        acc_v, [r_v, s_v], ones, mask=in_a & (lanes == l))
                return 0

            lax.fori_loop(0, ng, adj_pass, 0)
            pltpu.sync_copy(acc_v, adj_hbm.at[pl.ds(plo, half)])

        pltpu.sync_copy(c_v, cnt_hbm.at[pl.ds(row0, per_rows)])

    return k(src, dst, etype)


def _sc_edge_scores(ab, src, dst):
    E = src.shape[0]
    W = ab.shape[1]
    per_w = E // N_WORKERS
    chunk = min(per_w, 96)
    nchunks = per_w // chunk
    mesh = plsc.VectorSubcoreMesh(core_axis_name="c", subcore_axis_name="s")

    cp = pltpu.CompilerParams()
    if "needs_layout_passes" in pltpu.CompilerParams.__dataclass_fields__:
        cp = dataclasses.replace(cp, needs_layout_passes=False)

    @functools.partial(
        pl.kernel, mesh=mesh, compiler_params=cp,
        out_type=jax.ShapeDtypeStruct((E, 16), jnp.float32),
        scratch_types=[
            pltpu.VMEM((chunk,), jnp.int32),
            pltpu.VMEM((chunk,), jnp.int32),
            pltpu.VMEM((chunk, W), jnp.float32),
            pltpu.VMEM((chunk, W), jnp.float32),
            pltpu.VMEM((chunk, 16), jnp.float32),
            pltpu.SemaphoreType.DMA,
            pltpu.SemaphoreType.DMA,
        ])
    def k(ab_hbm, src_hbm, dst_hbm, out_hbm,
          src_v, dst_v, arow_v, brow_v, o_v, sem1, sem2):
        wid = lax.axis_index("s") * N_SC_CORES + lax.axis_index("c")
        base = wid * per_w
        lanes = lax.iota(jnp.int32, 16)

        @pl.loop(0, nchunks)
        def _(c):
            b = base + c * chunk
            pltpu.sync_copy(src_hbm.at[pl.ds(b, chunk)], src_v)
            pltpu.sync_copy(dst_hbm.at[pl.ds(b, chunk)], dst_v)
            cp1 = pltpu.async_copy(ab_hbm.at[src_v], arow_v, sem1)
            cp2 = pltpu.async_copy(ab_hbm.at[dst_v], brow_v, sem2)
            cp1.wait()
            cp2.wait()

            @pl.loop(0, chunk)
            def _(i):
                row = jnp.full((16,), 0, jnp.int32) + i
                a = plsc.load_gather(arow_v, [row, lanes])
                bb = plsc.load_gather(brow_v, [row, lanes + 16])
                plsc.store_scatter(o_v, [row, lanes], a + bb)

            pltpu.sync_copy(o_v, out_hbm.at[pl.ds(b, chunk)])

    return k(ab, src, dst)


def _flash_layer(s_arr, s_col, agg, wq, kv, k_col, v_col,
                 yb3, xb3, sched, D, BY=128, BX=512,
                 out_dtype=jnp.float32):
    NYr = agg.shape[0]
    NXr = kv.shape[0]
    ny = NYr // BY
    nx = NXr // BX
    scale = 1.0 / math.sqrt(float(D))
    kb_col = k_col // D
    vb_col = v_col // D
    sb_col = s_col // D

    def body(sched_ref, s_ref, a_ref, wq_ref, k_ref, v_ref,
             yb_ref, xb_ref, o_ref, h_s, q_s, acc_s, m_s, l_s):
        i = pl.program_id(0)
        j = pl.program_id(1)

        @pl.when(j == 0)
        def _():
            h = s_ref[...] + a_ref[...]
            h_s[...] = h
            q_s[...] = (jnp.dot(h.astype(jnp.bfloat16), wq_ref[...],
                                preferred_element_type=jnp.float32)
                        * scale).astype(jnp.bfloat16)
            acc_s[...] = jnp.zeros_like(acc_s)
            m_s[...] = jnp.full_like(m_s, -jnp.inf)
            l_s[...] = jnp.zeros_like(l_s)

        lo = sched_ref[0, i]
        hi = sched_ref[1, i]

        @pl.when((j >= lo) & (j <= hi))
        def _():
            sc = lax.dot_general(
                q_s[...], k_ref[...], (((1,), (1,)), ((), ())),
                preferred_element_type=jnp.float32)
            yb = yb_ref[0]
            xb = xb_ref[0]
            mask = yb == xb
            sc = jnp.where(mask, sc, NEG_INF)
            m_prev = m_s[...]
            m_new = jnp.maximum(m_prev, jnp.max(sc, axis=1, keepdims=True))
            p = jnp.exp(sc - m_new)
            corr = jnp.exp(m_prev - m_new)
            l_s[...] = l_s[...] * corr + jnp.sum(p, axis=1, keepdims=True)
            m_s[...] = m_new
            acc_s[...] = acc_s[...] * corr + jnp.dot(
                p.astype(jnp.bfloat16), v_ref[...],
                preferred_element_type=jnp.float32)

        @pl.when(j == nx - 1)
        def _():
            o_ref[...] = jnp.maximum(
                h_s[...] + acc_s[...] / l_s[...], 0.0).astype(o_ref.dtype)

    def kv_index(col):
        def f(i, j, sched):
            return (jnp.clip(j, sched[0, i], sched[1, i]), col)
        return f

    def xb_index(i, j, sched):
        return (jnp.clip(j, sched[0, i], sched[1, i]), 0, 0)

    grid_spec = pltpu.PrefetchScalarGridSpec(
        num_scalar_prefetch=1,
        grid=(ny, nx),
        in_specs=[
            pl.BlockSpec((BY, D), lambda i, j, sched: (i, sb_col)),
            pl.BlockSpec((BY, D), lambda i, j, sched: (i, 0)),
            pl.BlockSpec((D, D), lambda i, j, sched: (0, 0)),
            pl.BlockSpec((BX, D), kv_index(kb_col)),
            pl.BlockSpec((BX, D), kv_index(vb_col)),
            pl.BlockSpec((1, BY, 1), lambda i, j, sched: (i, 0, 0)),
            pl.BlockSpec((1, 1, BX), xb_index),
        ],
        out_specs=pl.BlockSpec((BY, D), lambda i, j, sched: (i, 0)),
        scratch_shapes=[
            pltpu.VMEM((BY, D), jnp.float32),
            pltpu.VMEM((BY, D), jnp.bfloat16),
            pltpu.VMEM((BY, D), jnp.float32),
            pltpu.VMEM((BY, 1), jnp.float32),
            pltpu.VMEM((BY, 1), jnp.float32),
        ],
    )
    return pl.pallas_call(
        body,
        grid_spec=grid_spec,
        out_shape=jax.ShapeDtypeStruct((NYr, D), out_dtype),
        compiler_params=pltpu.CompilerParams(
            dimension_semantics=("parallel", "arbitrary")),
    )(sched, s_arr, agg, wq, kv, kv, yb3, xb3)


def _flash_layer2(s_arr, s_col, agg, wq, kv, k_col, v_col,
                  yb3, xb2, sched, D, BY=128, BX=512,
                  out_dtype=jnp.float32):
    NYr = agg.shape[0]
    ny = NYr // BY
    scale = 1.0 / math.sqrt(float(D))
    sb_col = s_col // D
    bf16 = jnp.bfloat16

    def body(sched_ref, s_ref, a_ref, wq_ref, kv_ref, yb_ref, xb_ref, o_ref,
             kbuf, vbuf, acc_s, m_s, l_s, ksem, vsem):
        i = pl.program_id(0)
        lo = sched_ref[0, i]
        hi = sched_ref[1, i]
        h = s_ref[...] + a_ref[...]
        q = (jnp.dot(h.astype(bf16), wq_ref[...],
                     preferred_element_type=jnp.float32) * scale).astype(bf16)
        yb = yb_ref[0]

        def start_copy(jx, slot):
            pltpu.make_async_copy(
                kv_ref.at[pl.ds(jx * BX, BX), pl.ds(k_col, D)],
                kbuf.at[slot], ksem.at[slot]).start()
            pltpu.make_async_copy(
                kv_ref.at[pl.ds(jx * BX, BX), pl.ds(v_col, D)],
                vbuf.at[slot], vsem.at[slot]).start()

        start_copy(lo, 0)
        acc = jnp.zeros((BY, D), jnp.float32)
        acc_s[...] = acc
        m_s[...] = jnp.full((BY, 1), -jnp.inf, jnp.float32)
        l_s[...] = jnp.zeros((BY, 1), jnp.float32)

        def step(j, slot):
            @pl.when(j < hi)
            def _():
                start_copy(j + 1, 1 - slot)

            pltpu.make_async_copy(
                kv_ref.at[pl.ds(j * BX, BX), pl.ds(k_col, D)],
                kbuf.at[slot], ksem.at[slot]).wait()
            pltpu.make_async_copy(
                kv_ref.at[pl.ds(j * BX, BX), pl.ds(v_col, D)],
                vbuf.at[slot], vsem.at[slot]).wait()
            sc = lax.dot_general(
                q, kbuf[slot], (((1,), (1,)), ((), ())),
                preferred_element_type=jnp.float32)
            xb = xb_ref[:, pl.ds(j * BX, BX)]
            sc = jnp.where(yb == xb, sc, NEG_INF)
            m_prev = m_s[...]
            m_new = jnp.maximum(m_prev, jnp.max(sc, axis=1, keepdims=True))
            p = jnp.exp(sc - m_new)
            corr = jnp.exp(m_prev - m_new)
            l_s[...] = l_s[...] * corr + jnp.sum(p, axis=1, keepdims=True)
            m_s[...] = m_new
            acc_s[...] = acc_s[...] * corr + jnp.dot(
                p.astype(bf16), vbuf[slot],
                preferred_element_type=jnp.float32)
            return 1 - slot

        lax.fori_loop(lo, hi + 1, step, 0)
        o_ref[...] = jnp.maximum(
            h + acc_s[...] / l_s[...], 0.0).astype(o_ref.dtype)

    grid_spec = pltpu.PrefetchScalarGridSpec(
        num_scalar_prefetch=1,
        grid=(ny,),
        in_specs=[
            pl.BlockSpec((BY, D), lambda i, sched: (i, sb_col)),
            pl.BlockSpec((BY, D), lambda i, sched: (i, 0)),
            pl.BlockSpec((D, D), lambda i, sched: (0, 0)),
            pl.BlockSpec(memory_space=pl.ANY),
            pl.BlockSpec((1, BY, 1), lambda i, sched: (i, 0, 0)),
            pl.BlockSpec((1, xb2.shape[1]), lambda i, sched: (0, 0)),
        ],
        out_specs=pl.BlockSpec((BY, D), lambda i, sched: (i, 0)),
        scratch_shapes=[
            pltpu.VMEM((2, BX, D), bf16),
            pltpu.VMEM((2, BX, D), bf16),
            pltpu.VMEM((BY, D), jnp.float32),
            pltpu.VMEM((BY, 1), jnp.float32),
            pltpu.VMEM((BY, 1), jnp.float32),
            pltpu.SemaphoreType.DMA((2,)),
            pltpu.SemaphoreType.DMA((2,)),
        ],
    )
    return pl.pallas_call(
        body,
        grid_spec=grid_spec,
        out_shape=jax.ShapeDtypeStruct((NYr, D), out_dtype),
        compiler_params=pltpu.CompilerParams(
            dimension_semantics=("parallel",)),
    )(sched, s_arr, agg, wq, kv, yb3, xb2)


def _flash_layer3(y_prev, w1, A, B, cnt, embp, wq, kv, k_col, v_col,
                  yb3, xb2, sched, D, BY=256, BX=512,
                  out_dtype=jnp.float32):
    NYr = A.shape[0]
    KA = A.shape[1]
    in_d = y_prev.shape[1]
    ny = NYr // BY
    scale = 1.0 / math.sqrt(float(D))
    bf16 = jnp.bfloat16

    def body(sched_ref, y_ref, w1_ref, a_ref, b_ref, cnt_ref, embp_ref,
             wq_ref, kv_ref, yb_ref, xb_ref, o_ref,
             kbuf, vbuf, acc_s, m_s, l_s, ksem, vsem):
        i = pl.program_id(0)
        lo = sched_ref[0, i]
        hi = sched_ref[1, i]

        def start_copy(jx, slot):
            pltpu.make_async_copy(
                kv_ref.at[pl.ds(jx * BX, BX), pl.ds(k_col, D)],
                kbuf.at[slot], ksem.at[slot]).start()
            pltpu.make_async_copy(
                kv_ref.at[pl.ds(jx * BX, BX), pl.ds(v_col, D)],
                vbuf.at[slot], vsem.at[slot]).start()

        start_copy(lo, 0)
        h = (jnp.dot(y_ref[...].astype(bf16), w1_ref[...],
                     preferred_element_type=jnp.float32)
             + jnp.dot(a_ref[...].astype(bf16), b_ref[...],
                       preferred_element_type=jnp.float32)
             + jnp.dot(cnt_ref[...].astype(bf16), embp_ref[...],
                       preferred_element_type=jnp.float32))
        q = (jnp.dot(h.astype(bf16), wq_ref[...],
                     preferred_element_type=jnp.float32) * scale).astype(bf16)
        yb = yb_ref[0]

        acc_s[...] = jnp.zeros((BY, D), jnp.float32)
        m_s[...] = jnp.full((BY, 1), -jnp.inf, jnp.float32)
        l_s[...] = jnp.zeros((BY, 1), jnp.float32)

        def step(j, slot):
            @pl.when(j < hi)
            def _():
                start_copy(j + 1, 1 - slot)

            pltpu.make_async_copy(
                kv_ref.at[pl.ds(j * BX, BX), pl.ds(k_col, D)],
                kbuf.at[slot], ksem.at[slot]).wait()
            pltpu.make_async_copy(
                kv_ref.at[pl.ds(j * BX, BX), pl.ds(v_col, D)],
                vbuf.at[slot], vsem.at[slot]).wait()
            sc = lax.dot_general(
                q, kbuf[slot], (((1,), (1,)), ((), ())),
                preferred_element_type=jnp.float32)
            xb = xb_ref[:, pl.ds(j * BX, BX)]
            sc = jnp.where(yb == xb, sc, NEG_INF)
            m_prev = m_s[...]
            m_new = jnp.maximum(m_prev, jnp.max(sc, axis=1, keepdims=True))
            p = jnp.exp(sc - m_new)
            corr = jnp.exp(m_prev - m_new)
            l_s[...] = l_s[...] * corr + jnp.sum(p, axis=1, keepdims=True)
            m_s[...] = m_new
            acc_s[...] = acc_s[...] * corr + jnp.dot(
                p.astype(bf16), vbuf[slot],
                preferred_element_type=jnp.float32)
            return 1 - slot

        lax.fori_loop(lo, hi + 1, step, 0)
        o_ref[...] = jnp.maximum(
            h + acc_s[...] / l_s[...], 0.0).astype(o_ref.dtype)

    grid_spec = pltpu.PrefetchScalarGridSpec(
        num_scalar_prefetch=1,
        grid=(ny,),
        in_specs=[
            pl.BlockSpec((BY, in_d), lambda i, sched: (i, 0)),
            pl.BlockSpec((in_d, D), lambda i, sched: (0, 0)),
            pl.BlockSpec((BY, KA), lambda i, sched: (i, 0)),
            pl.BlockSpec((KA, D), lambda i, sched: (0, 0)),
            pl.BlockSpec((BY, 16), lambda i, sched: (i, 0)),
            pl.BlockSpec((16, D), lambda i, sched: (0, 0)),
            pl.BlockSpec((D, D), lambda i, sched: (0, 0)),
            pl.BlockSpec(memory_space=pl.ANY),
            pl.BlockSpec((1, BY, 1), lambda i, sched: (i, 0, 0)),
            pl.BlockSpec((1, xb2.shape[1]), lambda i, sched: (0, 0)),
        ],
        out_specs=pl.BlockSpec((BY, D), lambda i, sched: (i, 0)),
        scratch_shapes=[
            pltpu.VMEM((2, BX, D), bf16),
            pltpu.VMEM((2, BX, D), bf16),
            pltpu.VMEM((BY, D), jnp.float32),
            pltpu.VMEM((BY, 1), jnp.float32),
            pltpu.VMEM((BY, 1), jnp.float32),
            pltpu.SemaphoreType.DMA((2,)),
            pltpu.SemaphoreType.DMA((2,)),
        ],
    )
    return pl.pallas_call(
        body,
        grid_spec=grid_spec,
        out_shape=jax.ShapeDtypeStruct((NYr, D), out_dtype),
        compiler_params=pltpu.CompilerParams(
            dimension_semantics=("parallel",)),
    )(sched, y_prev, w1, A, B, cnt, embp, wq, kv, yb3, xb2)


def _block_schedule(y_batch, x_batch, BY, BX, nx):
    ny = y_batch.shape[0] // BY
    b_lo = y_batch[::BY]
    b_hi = y_batch[BY - 1::BY]
    bounds = jnp.searchsorted(x_batch, jnp.arange(NUM_BATCHES + 1),
                              side='left').astype(jnp.int32)
    xs = bounds[b_lo]
    xe = bounds[b_hi + 1]
    lo = jnp.clip(xs // BX, 0, nx - 1)
    hi = jnp.clip(jnp.maximum((xe - 1) // BX, lo), 0, nx - 1)
    return jnp.stack([lo, hi]).astype(jnp.int32)


def kernel(x, x_batch, tgt_y, tgt_edge_index, tgt_edge_type, tgt_y_batch,
           params):
    p = params
    g1, g2, g3 = p['gcn1'], p['gcn2'], p['gcn3']
    src = tgt_edge_index[0]
    dst = tgt_edge_index[1]
    H1 = g1['W_self'].shape[1]
    H2 = g2['W_self'].shape[1]
    H3 = g3['W_self'].shape[1]

    BY, BX = 512, 512
    NXr = x.shape[0]
    nx = NXr // BX
    ny = tgt_y_batch.shape[0] // BY
    sched = _block_schedule(tgt_y_batch, x_batch, BY, BX, nx)
    yb3 = tgt_y_batch.reshape(ny, BY, 1)
    xb2 = x_batch.reshape(1, NXr)

    bf16 = jnp.bfloat16

    kvw = jnp.concatenate(
        [g1['Wk'], g1['Wv'], g2['Wk'], g2['Wv'], g3['Wk'], g3['Wv']], axis=1)
    kv = _matmul(x, kvw, out_dtype=bf16, bm=2048)

    ny_nodes = tgt_y.shape[0]
    adj, cnt = _sc_build_adj(src, dst, tgt_edge_type, ny_nodes)

    def _emb_pad(emb):
        return jnp.zeros((16, emb.shape[1]), bf16).at[
            :emb.shape[0]].set(emb.astype(bf16))

    y0 = _sc_gather_rows(p['embed'], tgt_y)
    ay0 = _matmul(adj, y0, out_dtype=bf16)
    y1 = _flash_layer3(y0, g1['W_self'].astype(bf16), ay0,
                       g1['W_nb'].astype(bf16), cnt, _emb_pad(g1['edge_emb']),
                       g1['Wq'].astype(bf16), kv, 0, H1,
                       yb3, xb2, sched, H1, BY, BX, out_dtype=bf16)

    m2 = _matmul(y1, g2['W_nb'], out_dtype=bf16)
    y2 = _flash_layer3(y1, g2['W_self'].astype(bf16), adj, m2, cnt,
                       _emb_pad(g2['edge_emb']), g2['Wq'].astype(bf16), kv,
                       2 * H1, 2 * H1 + H2, yb3, xb2, sched, H2, BY, BX,
                       out_dtype=bf16)

    m3 = _matmul(y2, g3['W_nb'], out_dtype=bf16)
    y3 = _flash_layer3(y2, g3['W_self'].astype(bf16), adj, m3, cnt,
                       _emb_pad(g3['edge_emb']), g3['Wq'].astype(bf16), kv,
                       2 * (H1 + H2), 2 * (H1 + H2) + H3,
                       yb3, xb2, sched, H3, BY, BX)

    emb_d = y3.shape[1]
    n_rel = p['Wg'].shape[1]
    vocab = p['Wz'].shape[1]
    wg_pad = jnp.zeros((emb_d, 128), jnp.float32)
    wg_pad = wg_pad.at[:, 0:n_rel].set(p['Wg'][:emb_d])
    wg_pad = wg_pad.at[:, 16:16 + n_rel].set(p['Wg'][emb_d:])
    bg_pad = jnp.zeros((128,), jnp.float32).at[0:n_rel].set(p['bg'])
    head_w = jnp.concatenate([p['Wz'], wg_pad], axis=1)
    head_b = jnp.concatenate([p['bz'], bg_pad])
    head = _matmul(y3, head_w, bias=head_b, bn=vocab + 128)
    y_score = lax.slice_in_dim(head, 0, vocab, axis=1)
    ab = lax.slice_in_dim(head, vocab, vocab + 128, axis=1)
    er = _sc_edge_scores(ab, src, dst)
    y_edge_rel_score = lax.slice_in_dim(er, 0, n_rel, axis=1)

    return (y3, tgt_y_batch, tgt_edge_index, tgt_edge_type, y_score,
            y_edge_rel_score)


# speedup vs baseline: 1.8420x; 1.0719x over previous
import dataclasses
import functools
import math

import jax
import jax.numpy as jnp
from jax import lax
from jax.experimental import pallas as pl
from jax.experimental.pallas import tpu as pltpu
from jax.experimental.pallas import tpu_sc as plsc

N_SC_CORES = 2
N_SUBCORES = 16
N_WORKERS = N_SC_CORES * N_SUBCORES
NUM_BATCHES = 16
NEG_INF = -1e9


def _mm_body(a_ref, b_ref, o_ref):
    o_ref[...] = jax.lax.dot_general(
        a_ref[...].astype(jnp.bfloat16), b_ref[...].astype(jnp.bfloat16),
        (((1,), (0,)), ((), ())),
        preferred_element_type=jnp.float32).astype(o_ref.dtype)


def _mm_bias_body(a_ref, b_ref, bias_ref, o_ref):
    o_ref[...] = (jax.lax.dot_general(
        a_ref[...].astype(jnp.bfloat16), b_ref[...].astype(jnp.bfloat16),
        (((1,), (0,)), ((), ())),
        preferred_element_type=jnp.float32)
        + bias_ref[...]).astype(o_ref.dtype)


def _mm2_body(a1_ref, b1_ref, a2_ref, b2_ref, o_ref):
    o_ref[...] = (jax.lax.dot_general(
        a1_ref[...], b1_ref[...], (((1,), (0,)), ((), ())),
        preferred_element_type=jnp.float32) + jax.lax.dot_general(
        a2_ref[...], b2_ref[...], (((1,), (0,)), ((), ())),
        preferred_element_type=jnp.float32)).astype(o_ref.dtype)


def _mm2(a1, b1, a2, b2, bm=512, bn=512, out_dtype=jnp.float32):
    M, K1 = a1.shape
    _, N = b1.shape
    K2 = a2.shape[1]
    bm = min(bm, M)
    bn = min(bn, N)
    grid = (M // bm, N // bn)
    return pl.pallas_call(
        _mm2_body,
        grid=grid,
        in_specs=[
            pl.BlockSpec((bm, K1), lambda i, j: (i, 0)),
            pl.BlockSpec((K1, bn), lambda i, j: (0, j)),
            pl.BlockSpec((bm, K2), lambda i, j: (i, 0)),
            pl.BlockSpec((K2, bn), lambda i, j: (0, j)),
        ],
        out_specs=pl.BlockSpec((bm, bn), lambda i, j: (i, j)),
        out_shape=jax.ShapeDtypeStruct((M, N), out_dtype),
        compiler_params=pltpu.CompilerParams(
            dimension_semantics=("parallel", "parallel")),
    )(a1, b1, a2, b2)


def _matmul(a, b, bias=None, bm=512, bn=512, out_dtype=jnp.float32):
    M, K = a.shape
    _, N = b.shape
    bm = min(bm, M)
    bn = min(bn, N)
    grid = (M // bm, N // bn)
    in_specs = [
        pl.BlockSpec((bm, K), lambda i, j: (i, 0)),
        pl.BlockSpec((K, bn), lambda i, j: (0, j)),
    ]
    args = [a, b]
    body = _mm_body
    if bias is not None:
        in_specs.append(pl.BlockSpec((1, bn), lambda i, j: (0, j)))
        args.append(bias.reshape(1, N))
        body = _mm_bias_body
    return pl.pallas_call(
        body,
        grid=grid,
        in_specs=in_specs,
        out_specs=pl.BlockSpec((bm, bn), lambda i, j: (i, j)),
        out_shape=jax.ShapeDtypeStruct((M, N), out_dtype),
        compiler_params=pltpu.CompilerParams(
            dimension_semantics=("parallel", "parallel")),
    )(*args)


def _sc_gather_rows(table, idx):
    n = idx.shape[0]
    _, D = table.shape
    per_w = n // N_WORKERS
    chunk = min(per_w, 64)
    nchunks = per_w // chunk
    mesh = plsc.VectorSubcoreMesh(core_axis_name="c", subcore_axis_name="s")

    @functools.partial(
        pl.kernel, mesh=mesh,
        out_type=jax.ShapeDtypeStruct((n, D), jnp.float32),
        scratch_types=[
            pltpu.VMEM((chunk,), jnp.int32),
            pltpu.VMEM((chunk, D), jnp.float32),
            pltpu.SemaphoreType.DMA,
        ])
    def k(table_hbm, idx_hbm, out_hbm, idx_v, rows_v, sem):
        wid = lax.axis_index("s") * N_SC_CORES + lax.axis_index("c")
        base = wid * per_w

        @pl.loop(0, nchunks)
        def _(c):
            b = base + c * chunk
            pltpu.sync_copy(idx_hbm.at[pl.ds(b, chunk)], idx_v)
            pltpu.async_copy(table_hbm.at[idx_v], rows_v, sem).wait()
            pltpu.sync_copy(rows_v, out_hbm.at[pl.ds(b, chunk)])

    return k(table, idx)


def _sc_build_adj(src, dst, etype, ny):
    E = src.shape[0]
    per_rows = ny // N_WORKERS
    half = per_rows // 2
    mesh = plsc.VectorSubcoreMesh(core_axis_name="c", subcore_axis_name="s")

    cp = pltpu.CompilerParams()
    if "needs_layout_passes" in pltpu.CompilerParams.__dataclass_fields__:
        cp = dataclasses.replace(cp, needs_layout_passes=False)

    @functools.partial(
        pl.kernel, mesh=mesh, compiler_params=cp,
        out_type=(jax.ShapeDtypeStruct((ny, ny), jnp.float32),
                  jax.ShapeDtypeStruct((ny, 16), jnp.float32)),
        scratch_types=[
            pltpu.VMEM((E,), jnp.int32),
            pltpu.VMEM((E,), jnp.int32),
            pltpu.VMEM((E,), jnp.int32),
            pltpu.VMEM((E + 16,), jnp.int32),
            pltpu.VMEM((E + 16,), jnp.int32),
            pltpu.VMEM((E + 16,), jnp.int32),
            pltpu.VMEM((half, ny), jnp.float32),
            pltpu.VMEM((per_rows, 16), jnp.float32),
        ])
    def k(src_hbm, dst_hbm, et_hbm, adj_hbm, cnt_hbm,
          src_v, dst_v, et_v, cs_v, cd_v, ct_v, acc_v, c_v):
        cid = lax.axis_index("c")
        sid = lax.axis_index("s")
        wid = sid * N_SC_CORES + cid
        row0 = wid * per_rows
        pltpu.sync_copy(src_hbm, src_v)
        pltpu.sync_copy(dst_hbm, dst_v)
        pltpu.sync_copy(et_hbm, et_v)

        @pl.loop(0, per_rows)
        def _(r):
            c_v.at[r, pl.ds(0, 16)][...] = jnp.zeros((16,), jnp.float32)

        lanes = lax.iota(jnp.int32, 16)
        ones = jnp.full((16,), 1.0, jnp.float32)

        def compress(g, off):
            b = g * 16
            d_v = dst_v[pl.ds(b, 16)]
            m = (d_v >= row0) & (d_v < row0 + per_rows)
            plsc.store_compressed(cd_v.at[pl.ds(off, 16)], d_v, mask=m)
            plsc.store_compressed(cs_v.at[pl.ds(off, 16)],
                                  src_v[pl.ds(b, 16)], mask=m)
            plsc.store_compressed(ct_v.at[pl.ds(off, 16)],
                                  et_v[pl.ds(b, 16)], mask=m)
            return off + jnp.sum(m.astype(jnp.int32), axis=0)

        nmine = lax.fori_loop(0, E // 16, compress, jnp.int32(0))
        ng = (nmine + 15) // 16

        def cnt_pass(g, _):
            b = g * 16
            valid = lanes < (nmine - b)
            d_v = cd_v[pl.ds(b, 16)]
            t_v = ct_v[pl.ds(b, 16)]
            rc_v = jnp.clip(d_v - row0, 0, per_rows - 1)
            for l in range(16):
                plsc.addupdate_scatter(
                    c_v, [rc_v, t_v], ones, mask=valid & (lanes == l))
            return 0

        lax.fori_loop(0, ng, cnt_pass, 0)

        for p in range(2):
            plo = row0 + p * half

            @pl.loop(0, half)
            def _(r):
                @pl.loop(0, ny, step=16)
                def _(cc):
                    acc_v.at[r, pl.ds(cc, 16)][...] = jnp.zeros(
                        (16,), jnp.float32)

            def adj_pass(g, _):
                b = g * 16
                valid = lanes < (nmine - b)
                d_v = cd_v[pl.ds(b, 16)]
                s_v = cs_v[pl.ds(b, 16)]
                in_a = valid & (d_v >= plo) & (d_v < plo + half)
                r_v = jnp.clip(d_v - plo, 0, half - 1)
                for l in range(16):
                    plsc.addupdate_scatter(
                        acc_v, [r_v, s_v], ones, mask=in_a & (lanes == l))
                return 0

            lax.fori_loop(0, ng, adj_pass, 0)
            pltpu.sync_copy(acc_v, adj_hbm.at[pl.ds(plo, half)])

        pltpu.sync_copy(c_v, cnt_hbm.at[pl.ds(row0, per_rows)])

    return k(src, dst, etype)


def _sc_edge_scores(ab, src, dst):
    E = src.shape[0]
    W = ab.shape[1]
    per_w = E // N_WORKERS
    chunk = min(per_w, 96)
    nchunks = per_w // chunk
    mesh = plsc.VectorSubcoreMesh(core_axis_name="c", subcore_axis_name="s")

    cp = pltpu.CompilerParams()
    if "needs_layout_passes" in pltpu.CompilerParams.__dataclass_fields__:
        cp = dataclasses.replace(cp, needs_layout_passes=False)

    @functools.partial(
        pl.kernel, mesh=mesh, compiler_params=cp,
        out_type=jax.ShapeDtypeStruct((E, 16), jnp.float32),
        scratch_types=[
            pltpu.VMEM((chunk,), jnp.int32),
            pltpu.VMEM((chunk,), jnp.int32),
            pltpu.VMEM((chunk, W), jnp.float32),
            pltpu.VMEM((chunk, W), jnp.float32),
            pltpu.VMEM((chunk, 16), jnp.float32),
            pltpu.SemaphoreType.DMA,
            pltpu.SemaphoreType.DMA,
        ])
    def k(ab_hbm, src_hbm, dst_hbm, out_hbm,
          src_v, dst_v, arow_v, brow_v, o_v, sem1, sem2):
        wid = lax.axis_index("s") * N_SC_CORES + lax.axis_index("c")
        base = wid * per_w
        lanes = lax.iota(jnp.int32, 16)

        @pl.loop(0, nchunks)
        def _(c):
            b = base + c * chunk
            pltpu.sync_copy(src_hbm.at[pl.ds(b, chunk)], src_v)
            pltpu.sync_copy(dst_hbm.at[pl.ds(b, chunk)], dst_v)
            cp1 = pltpu.async_copy(ab_hbm.at[src_v], arow_v, sem1)
            cp2 = pltpu.async_copy(ab_hbm.at[dst_v], brow_v, sem2)
            cp1.wait()
            cp2.wait()

            @pl.loop(0, chunk)
            def _(i):
                row = jnp.full((16,), 0, jnp.int32) + i
                a = plsc.load_gather(arow_v, [row, lanes])
                bb = plsc.load_gather(brow_v, [row, lanes + 16])
                plsc.store_scatter(o_v, [row, lanes], a + bb)

            pltpu.sync_copy(o_v, out_hbm.at[pl.ds(b, chunk)])

    return k(ab, src, dst)


def _flash_layer(s_arr, s_col, agg, wq, kv, k_col, v_col,
                 yb3, xb3, sched, D, BY=128, BX=512,
                 out_dtype=jnp.float32):
    NYr = agg.shape[0]
    NXr = kv.shape[0]
    ny = NYr // BY
    nx = NXr // BX
    scale = 1.0 / math.sqrt(float(D))
    kb_col = k_col // D
    vb_col = v_col // D
    sb_col = s_col // D

    def body(sched_ref, s_ref, a_ref, wq_ref, k_ref, v_ref,
             yb_ref, xb_ref, o_ref, h_s, q_s, acc_s, m_s, l_s):
        i = pl.program_id(0)
        j = pl.program_id(1)

        @pl.when(j == 0)
        def _():
            h = s_ref[...] + a_ref[...]
            h_s[...] = h
            q_s[...] = (jnp.dot(h.astype(jnp.bfloat16), wq_ref[...],
                                preferred_element_type=jnp.float32)
                        * scale).astype(jnp.bfloat16)
            acc_s[...] = jnp.zeros_like(acc_s)
            m_s[...] = jnp.full_like(m_s, -jnp.inf)
            l_s[...] = jnp.zeros_like(l_s)

        lo = sched_ref[0, i]
        hi = sched_ref[1, i]

        @pl.when((j >= lo) & (j <= hi))
        def _():
            sc = lax.dot_general(
                q_s[...], k_ref[...], (((1,), (1,)), ((), ())),
                preferred_element_type=jnp.float32)
            yb = yb_ref[0]
            xb = xb_ref[0]
            mask = yb == xb
            sc = jnp.where(mask, sc, NEG_INF)
            m_prev = m_s[...]
            m_new = jnp.maximum(m_prev, jnp.max(sc, axis=1, keepdims=True))
            p = jnp.exp(sc - m_new)
            corr = jnp.exp(m_prev - m_new)
            l_s[...] = l_s[...] * corr + jnp.sum(p, axis=1, keepdims=True)
            m_s[...] = m_new
            acc_s[...] = acc_s[...] * corr + jnp.dot(
                p.astype(jnp.bfloat16), v_ref[...],
                preferred_element_type=jnp.float32)

        @pl.when(j == nx - 1)
        def _():
            o_ref[...] = jnp.maximum(
                h_s[...] + acc_s[...] / l_s[...], 0.0).astype(o_ref.dtype)

    def kv_index(col):
        def f(i, j, sched):
            return (jnp.clip(j, sched[0, i], sched[1, i]), col)
        return f

    def xb_index(i, j, sched):
        return (jnp.clip(j, sched[0, i], sched[1, i]), 0, 0)

    grid_spec = pltpu.PrefetchScalarGridSpec(
        num_scalar_prefetch=1,
        grid=(ny, nx),
        in_specs=[
            pl.BlockSpec((BY, D), lambda i, j, sched: (i, sb_col)),
            pl.BlockSpec((BY, D), lambda i, j, sched: (i, 0)),
            pl.BlockSpec((D, D), lambda i, j, sched: (0, 0)),
            pl.BlockSpec((BX, D), kv_index(kb_col)),
            pl.BlockSpec((BX, D), kv_index(vb_col)),
            pl.BlockSpec((1, BY, 1), lambda i, j, sched: (i, 0, 0)),
            pl.BlockSpec((1, 1, BX), xb_index),
        ],
        out_specs=pl.BlockSpec((BY, D), lambda i, j, sched: (i, 0)),
        scratch_shapes=[
            pltpu.VMEM((BY, D), jnp.float32),
            pltpu.VMEM((BY, D), jnp.bfloat16),
            pltpu.VMEM((BY, D), jnp.float32),
            pltpu.VMEM((BY, 1), jnp.float32),
            pltpu.VMEM((BY, 1), jnp.float32),
        ],
    )
    return pl.pallas_call(
        body,
        grid_spec=grid_spec,
        out_shape=jax.ShapeDtypeStruct((NYr, D), out_dtype),
        compiler_params=pltpu.CompilerParams(
            dimension_semantics=("parallel", "arbitrary")),
    )(sched, s_arr, agg, wq, kv, kv, yb3, xb3)


def _flash_layer2(s_arr, s_col, agg, wq, kv, k_col, v_col,
                  yb3, xb2, sched, D, BY=128, BX=512,
                  out_dtype=jnp.float32):
    NYr = agg.shape[0]
    ny = NYr // BY
    scale = 1.0 / math.sqrt(float(D))
    sb_col = s_col // D
    bf16 = jnp.bfloat16

    def body(sched_ref, s_ref, a_ref, wq_ref, kv_ref, yb_ref, xb_ref, o_ref,
             kbuf, vbuf, acc_s, m_s, l_s, ksem, vsem):
        i = pl.program_id(0)
        lo = sched_ref[0, i]
        hi = sched_ref[1, i]
        h = s_ref[...] + a_ref[...]
        q = (jnp.dot(h.astype(bf16), wq_ref[...],
                     preferred_element_type=jnp.float32) * scale).astype(bf16)
        yb = yb_ref[0]

        def start_copy(jx, slot):
            pltpu.make_async_copy(
                kv_ref.at[pl.ds(jx * BX, BX), pl.ds(k_col, D)],
                kbuf.at[slot], ksem.at[slot]).start()
            pltpu.make_async_copy(
                kv_ref.at[pl.ds(jx * BX, BX), pl.ds(v_col, D)],
                vbuf.at[slot], vsem.at[slot]).start()

        start_copy(lo, 0)
        acc = jnp.zeros((BY, D), jnp.float32)
        acc_s[...] = acc
        m_s[...] = jnp.full((BY, 1), -jnp.inf, jnp.float32)
        l_s[...] = jnp.zeros((BY, 1), jnp.float32)

        def step(j, slot):
            @pl.when(j < hi)
            def _():
                start_copy(j + 1, 1 - slot)

            pltpu.make_async_copy(
                kv_ref.at[pl.ds(j * BX, BX), pl.ds(k_col, D)],
                kbuf.at[slot], ksem.at[slot]).wait()
            pltpu.make_async_copy(
                kv_ref.at[pl.ds(j * BX, BX), pl.ds(v_col, D)],
                vbuf.at[slot], vsem.at[slot]).wait()
            sc = lax.dot_general(
                q, kbuf[slot], (((1,), (1,)), ((), ())),
                preferred_element_type=jnp.float32)
            xb = xb_ref[:, pl.ds(j * BX, BX)]
            sc = jnp.where(yb == xb, sc, NEG_INF)
            m_prev = m_s[...]
            m_new = jnp.maximum(m_prev, jnp.max(sc, axis=1, keepdims=True))
            p = jnp.exp(sc - m_new)
            corr = jnp.exp(m_prev - m_new)
            l_s[...] = l_s[...] * corr + jnp.sum(p, axis=1, keepdims=True)
            m_s[...] = m_new
            acc_s[...] = acc_s[...] * corr + jnp.dot(
                p.astype(bf16), vbuf[slot],
                preferred_element_type=jnp.float32)
            return 1 - slot

        lax.fori_loop(lo, hi + 1, step, 0)
        o_ref[...] = jnp.maximum(
            h + acc_s[...] / l_s[...], 0.0).astype(o_ref.dtype)

    grid_spec = pltpu.PrefetchScalarGridSpec(
        num_scalar_prefetch=1,
        grid=(ny,),
        in_specs=[
            pl.BlockSpec((BY, D), lambda i, sched: (i, sb_col)),
            pl.BlockSpec((BY, D), lambda i, sched: (i, 0)),
            pl.BlockSpec((D, D), lambda i, sched: (0, 0)),
            pl.BlockSpec(memory_space=pl.ANY),
            pl.BlockSpec((1, BY, 1), lambda i, sched: (i, 0, 0)),
            pl.BlockSpec((1, xb2.shape[1]), lambda i, sched: (0, 0)),
        ],
        out_specs=pl.BlockSpec((BY, D), lambda i, sched: (i, 0)),
        scratch_shapes=[
            pltpu.VMEM((2, BX, D), bf16),
            pltpu.VMEM((2, BX, D), bf16),
            pltpu.VMEM((BY, D), jnp.float32),
            pltpu.VMEM((BY, 1), jnp.float32),
            pltpu.VMEM((BY, 1), jnp.float32),
            pltpu.SemaphoreType.DMA((2,)),
            pltpu.SemaphoreType.DMA((2,)),
        ],
    )
    return pl.pallas_call(
        body,
        grid_spec=grid_spec,
        out_shape=jax.ShapeDtypeStruct((NYr, D), out_dtype),
        compiler_params=pltpu.CompilerParams(
            dimension_semantics=("parallel",)),
    )(sched, s_arr, agg, wq, kv, yb3, xb2)


def _flash_layer3(y_prev, w1, A, B, cnt, embp, wq, kv, k_col, v_col,
                  yb3, xb2, sched, D, BY=256, BX=512,
                  out_dtype=jnp.float32):
    NYr = A.shape[0]
    KA = A.shape[1]
    in_d = y_prev.shape[1]
    ny = NYr // BY
    scale = 1.0 / math.sqrt(float(D))
    bf16 = jnp.bfloat16

    def body(sched_ref, y_ref, w1_ref, a_ref, b_ref, cnt_ref, embp_ref,
             wq_ref, kv_ref, yb_ref, xb_ref, o_ref,
             kbuf, vbuf, acc_s, m_s, l_s, ksem, vsem):
        i = pl.program_id(0)
        lo = sched_ref[0, i]
        hi = sched_ref[1, i]

        def start_copy(jx, slot):
            pltpu.make_async_copy(
                kv_ref.at[pl.ds(jx * BX, BX), pl.ds(k_col, D)],
                kbuf.at[slot], ksem.at[slot]).start()
            pltpu.make_async_copy(
                kv_ref.at[pl.ds(jx * BX, BX), pl.ds(v_col, D)],
                vbuf.at[slot], vsem.at[slot]).start()

        start_copy(lo, 0)
        h = (jnp.dot(y_ref[...].astype(bf16), w1_ref[...],
                     preferred_element_type=jnp.float32)
             + jnp.dot(a_ref[...].astype(bf16), b_ref[...],
                       preferred_element_type=jnp.float32)
             + jnp.dot(cnt_ref[...].astype(bf16), embp_ref[...],
                       preferred_element_type=jnp.float32))
        q = (jnp.dot(h.astype(bf16), wq_ref[...],
                     preferred_element_type=jnp.float32) * scale).astype(bf16)
        yb = yb_ref[0]

        acc_s[...] = jnp.zeros((BY, D), jnp.float32)
        m_s[...] = jnp.full((BY, 1), -jnp.inf, jnp.float32)
        l_s[...] = jnp.zeros((BY, 1), jnp.float32)

        def step(j, slot):
            @pl.when(j < hi)
            def _():
                start_copy(j + 1, 1 - slot)

            pltpu.make_async_copy(
                kv_ref.at[pl.ds(j * BX, BX), pl.ds(k_col, D)],
                kbuf.at[slot], ksem.at[slot]).wait()
            pltpu.make_async_copy(
                kv_ref.at[pl.ds(j * BX, BX), pl.ds(v_col, D)],
                vbuf.at[slot], vsem.at[slot]).wait()
            sc = lax.dot_general(
                q, kbuf[slot], (((1,), (1,)), ((), ())),
                preferred_element_type=jnp.float32)
            xb = xb_ref[:, pl.ds(j * BX, BX)]
            sc = jnp.where(yb == xb, sc, NEG_INF)
            m_prev = m_s[...]
            m_new = jnp.maximum(m_prev, jnp.max(sc, axis=1, keepdims=True))
            p = jnp.exp(sc - m_new)
            corr = jnp.exp(m_prev - m_new)
            l_s[...] = l_s[...] * corr + jnp.sum(p, axis=1, keepdims=True)
            m_s[...] = m_new
            acc_s[...] = acc_s[...] * corr + jnp.dot(
                p.astype(bf16), vbuf[slot],
                preferred_element_type=jnp.float32)
            return 1 - slot

        lax.fori_loop(lo, hi + 1, step, 0)
        o_ref[...] = jnp.maximum(
            h + acc_s[...] / l_s[...], 0.0).astype(o_ref.dtype)

    grid_spec = pltpu.PrefetchScalarGridSpec(
        num_scalar_prefetch=1,
        grid=(ny,),
        in_specs=[
            pl.BlockSpec((BY, in_d), lambda i, sched: (i, 0)),
            pl.BlockSpec((in_d, D), lambda i, sched: (0, 0)),
            pl.BlockSpec((BY, KA), lambda i, sched: (i, 0)),
            pl.BlockSpec((KA, D), lambda i, sched: (0, 0)),
            pl.BlockSpec((BY, 16), lambda i, sched: (i, 0)),
            pl.BlockSpec((16, D), lambda i, sched: (0, 0)),
            pl.BlockSpec((D, D), lambda i, sched: (0, 0)),
            pl.BlockSpec(memory_space=pl.ANY),
            pl.BlockSpec((1, BY, 1), lambda i, sched: (i, 0, 0)),
            pl.BlockSpec((1, xb2.shape[1]), lambda i, sched: (0, 0)),
        ],
        out_specs=pl.BlockSpec((BY, D), lambda i, sched: (i, 0)),
        scratch_shapes=[
            pltpu.VMEM((2, BX, D), bf16),
            pltpu.VMEM((2, BX, D), bf16),
            pltpu.VMEM((BY, D), jnp.float32),
            pltpu.VMEM((BY, 1), jnp.float32),
            pltpu.VMEM((BY, 1), jnp.float32),
            pltpu.SemaphoreType.DMA((2,)),
            pltpu.SemaphoreType.DMA((2,)),
        ],
    )
    return pl.pallas_call(
        body,
        grid_spec=grid_spec,
        out_shape=jax.ShapeDtypeStruct((NYr, D), out_dtype),
        compiler_params=pltpu.CompilerParams(
            dimension_semantics=("parallel",)),
    )(sched, y_prev, w1, A, B, cnt, embp, wq, kv, yb3, xb2)


def _block_schedule(y_batch, x_batch, BY, BX, nx):
    ny = y_batch.shape[0] // BY
    b_lo = y_batch[::BY]
    b_hi = y_batch[BY - 1::BY]
    counts = jnp.sum(
        (x_batch[None, :] == jnp.arange(NUM_BATCHES)[:, None]).astype(
            jnp.int32), axis=1)
    bounds = jnp.concatenate(
        [jnp.zeros((1,), jnp.int32), jnp.cumsum(counts)]).astype(jnp.int32)
    xs = bounds[b_lo]
    xe = bounds[b_hi + 1]
    lo = jnp.clip(xs // BX, 0, nx - 1)
    hi = jnp.clip(jnp.maximum((xe - 1) // BX, lo), 0, nx - 1)
    return jnp.stack([lo, hi]).astype(jnp.int32)


def kernel(x, x_batch, tgt_y, tgt_edge_index, tgt_edge_type, tgt_y_batch,
           params):
    p = params
    g1, g2, g3 = p['gcn1'], p['gcn2'], p['gcn3']
    src = tgt_edge_index[0]
    dst = tgt_edge_index[1]
    H1 = g1['W_self'].shape[1]
    H2 = g2['W_self'].shape[1]
    H3 = g3['W_self'].shape[1]

    BY, BX = 512, 512
    NXr = x.shape[0]
    nx = NXr // BX
    ny = tgt_y_batch.shape[0] // BY
    sched = _block_schedule(tgt_y_batch, x_batch, BY, BX, nx)
    yb3 = tgt_y_batch.reshape(ny, BY, 1)
    xb2 = x_batch.reshape(1, NXr)

    bf16 = jnp.bfloat16

    kvw = jnp.concatenate(
        [g1['Wk'], g1['Wv'], g2['Wk'], g2['Wv'], g3['Wk'], g3['Wv']], axis=1)
    kv = _matmul(x, kvw, out_dtype=bf16, bm=2048, bn=kvw.shape[1])

    ny_nodes = tgt_y.shape[0]
    adj, cnt = _sc_build_adj(src, dst, tgt_edge_type, ny_nodes)

    def _emb_pad(emb):
        return jnp.concatenate(
            [emb.astype(bf16),
             jnp.zeros((16 - emb.shape[0], emb.shape[1]), bf16)], axis=0)

    y0 = _sc_gather_rows(p['embed'], tgt_y)
    ay0 = _matmul(adj, y0, out_dtype=bf16)
    y1 = _flash_layer3(y0, g1['W_self'].astype(bf16), ay0,
                       g1['W_nb'].astype(bf16), cnt, _emb_pad(g1['edge_emb']),
                       g1['Wq'].astype(bf16), kv, 0, H1,
                       yb3, xb2, sched, H1, BY, BX, out_dtype=bf16)

    m2 = _matmul(y1, g2['W_nb'], out_dtype=bf16)
    y2 = _flash_layer3(y1, g2['W_self'].astype(bf16), adj, m2, cnt,
                       _emb_pad(g2['edge_emb']), g2['Wq'].astype(bf16), kv,
                       2 * H1, 2 * H1 + H2, yb3, xb2, sched, H2, BY, BX,
                       out_dtype=bf16)

    m3 = _matmul(y2, g3['W_nb'], out_dtype=bf16)
    y3 = _flash_layer3(y2, g3['W_self'].astype(bf16), adj, m3, cnt,
                       _emb_pad(g3['edge_emb']), g3['Wq'].astype(bf16), kv,
                       2 * (H1 + H2), 2 * (H1 + H2) + H3,
                       yb3, xb2, sched, H3, BY, BX)

    emb_d = y3.shape[1]
    n_rel = p['Wg'].shape[1]
    y_score = _matmul(y3, p['Wz'], bias=p['bz'])
    wg_pad = jnp.concatenate(
        [jnp.pad(p['Wg'][:emb_d], ((0, 0), (0, 16 - n_rel))),
         jnp.pad(p['Wg'][emb_d:], ((0, 0), (0, 16 - n_rel))),
         jnp.zeros((emb_d, 96), jnp.float32)], axis=1)
    bg_pad = jnp.pad(p['bg'], (0, 128 - n_rel))
    ab = _matmul(y3, wg_pad, bias=bg_pad, bn=128)
    er = _sc_edge_scores(ab, src, dst)
    y_edge_rel_score = lax.slice_in_dim(er, 0, n_rel, axis=1)

    return (y3, tgt_y_batch, tgt_edge_index, tgt_edge_type, y_score,
            y_edge_rel_score)


# speedup vs baseline: 1.9283x; 1.0468x over previous
import dataclasses
import functools
import math

import jax
import jax.numpy as jnp
from jax import lax
from jax.experimental import pallas as pl
from jax.experimental.pallas import tpu as pltpu
from jax.experimental.pallas import tpu_sc as plsc

N_SC_CORES = 2
N_SUBCORES = 16
N_WORKERS = N_SC_CORES * N_SUBCORES
NUM_BATCHES = 16
NEG_INF = -1e9


def _mm_body(a_ref, b_ref, o_ref):
    o_ref[...] = jax.lax.dot_general(
        a_ref[...].astype(jnp.bfloat16), b_ref[...].astype(jnp.bfloat16),
        (((1,), (0,)), ((), ())),
        preferred_element_type=jnp.float32).astype(o_ref.dtype)


def _mm_bias_body(a_ref, b_ref, bias_ref, o_ref):
    o_ref[...] = (jax.lax.dot_general(
        a_ref[...].astype(jnp.bfloat16), b_ref[...].astype(jnp.bfloat16),
        (((1,), (0,)), ((), ())),
        preferred_element_type=jnp.float32)
        + bias_ref[...]).astype(o_ref.dtype)


def _mm2_body(a1_ref, b1_ref, a2_ref, b2_ref, o_ref):
    o_ref[...] = (jax.lax.dot_general(
        a1_ref[...], b1_ref[...], (((1,), (0,)), ((), ())),
        preferred_element_type=jnp.float32) + jax.lax.dot_general(
        a2_ref[...], b2_ref[...], (((1,), (0,)), ((), ())),
        preferred_element_type=jnp.float32)).astype(o_ref.dtype)


def _mm2(a1, b1, a2, b2, bm=512, bn=512, out_dtype=jnp.float32):
    M, K1 = a1.shape
    _, N = b1.shape
    K2 = a2.shape[1]
    bm = min(bm, M)
    bn = min(bn, N)
    grid = (M // bm, N // bn)
    return pl.pallas_call(
        _mm2_body,
        grid=grid,
        in_specs=[
            pl.BlockSpec((bm, K1), lambda i, j: (i, 0)),
            pl.BlockSpec((K1, bn), lambda i, j: (0, j)),
            pl.BlockSpec((bm, K2), lambda i, j: (i, 0)),
            pl.BlockSpec((K2, bn), lambda i, j: (0, j)),
        ],
        out_specs=pl.BlockSpec((bm, bn), lambda i, j: (i, j)),
        out_shape=jax.ShapeDtypeStruct((M, N), out_dtype),
        compiler_params=pltpu.CompilerParams(
            dimension_semantics=("parallel", "parallel")),
    )(a1, b1, a2, b2)


def _matmul(a, b, bias=None, bm=512, bn=512, out_dtype=jnp.float32):
    M, K = a.shape
    _, N = b.shape
    bm = min(bm, M)
    bn = min(bn, N)
    grid = (M // bm, N // bn)
    in_specs = [
        pl.BlockSpec((bm, K), lambda i, j: (i, 0)),
        pl.BlockSpec((K, bn), lambda i, j: (0, j)),
    ]
    args = [a, b]
    body = _mm_body
    if bias is not None:
        in_specs.append(pl.BlockSpec((1, bn), lambda i, j: (0, j)))
        args.append(bias.reshape(1, N))
        body = _mm_bias_body
    return pl.pallas_call(
        body,
        grid=grid,
        in_specs=in_specs,
        out_specs=pl.BlockSpec((bm, bn), lambda i, j: (i, j)),
        out_shape=jax.ShapeDtypeStruct((M, N), out_dtype),
        compiler_params=pltpu.CompilerParams(
            dimension_semantics=("parallel", "parallel")),
    )(*args)


def _sc_gather_rows(table, idx):
    n = idx.shape[0]
    _, D = table.shape
    per_w = n // N_WORKERS
    chunk = min(per_w, 64)
    nchunks = per_w // chunk
    mesh = plsc.VectorSubcoreMesh(core_axis_name="c", subcore_axis_name="s")

    @functools.partial(
        pl.kernel, mesh=mesh,
        out_type=jax.ShapeDtypeStruct((n, D), jnp.float32),
        scratch_types=[
            pltpu.VMEM((chunk,), jnp.int32),
            pltpu.VMEM((chunk, D), jnp.float32),
            pltpu.SemaphoreType.DMA,
        ])
    def k(table_hbm, idx_hbm, out_hbm, idx_v, rows_v, sem):
        wid = lax.axis_index("s") * N_SC_CORES + lax.axis_index("c")
        base = wid * per_w

        @pl.loop(0, nchunks)
        def _(c):
            b = base + c * chunk
            pltpu.sync_copy(idx_hbm.at[pl.ds(b, chunk)], idx_v)
            pltpu.async_copy(table_hbm.at[idx_v], rows_v, sem).wait()
            pltpu.sync_copy(rows_v, out_hbm.at[pl.ds(b, chunk)])

    return k(table, idx)


def _sc_build_adj(src, dst, etype, ny):
    E = src.shape[0]
    per_rows = ny // N_WORKERS
    half = per_rows // 2
    mesh = plsc.VectorSubcoreMesh(core_axis_name="c", subcore_axis_name="s")

    cp = pltpu.CompilerParams()
    if "needs_layout_passes" in pltpu.CompilerParams.__dataclass_fields__:
        cp = dataclasses.replace(cp, needs_layout_passes=False)

    @functools.partial(
        pl.kernel, mesh=mesh, compiler_params=cp,
        out_type=(jax.ShapeDtypeStruct((ny, ny), jnp.float32),
                  jax.ShapeDtypeStruct((ny, 16), jnp.float32)),
        scratch_types=[
            pltpu.VMEM((E,), jnp.int32),
            pltpu.VMEM((E,), jnp.int32),
            pltpu.VMEM((E,), jnp.int32),
            pltpu.VMEM((E + 16,), jnp.int32),
            pltpu.VMEM((E + 16,), jnp.int32),
            pltpu.VMEM((E + 16,), jnp.int32),
            pltpu.VMEM((half, ny), jnp.float32),
            pltpu.VMEM((per_rows, 16), jnp.float32),
        ])
    def k(src_hbm, dst_hbm, et_hbm, adj_hbm, cnt_hbm,
          src_v, dst_v, et_v, cs_v, cd_v, ct_v, acc_v, c_v):
        cid = lax.axis_index("c")
        sid = lax.axis_index("s")
        wid = sid * N_SC_CORES + cid
        row0 = wid * per_rows
        pltpu.sync_copy(src_hbm, src_v)
        pltpu.sync_copy(dst_hbm, dst_v)
        pltpu.sync_copy(et_hbm, et_v)

        @pl.loop(0, per_rows)
        def _(r):
            c_v.at[r, pl.ds(0, 16)][...] = jnp.zeros((16,), jnp.float32)

        lanes = lax.iota(jnp.int32, 16)
        ones = jnp.full((16,), 1.0, jnp.float32)

        def compress(g, off):
            b = g * 16
            d_v = dst_v[pl.ds(b, 16)]
            m = (d_v >= row0) & (d_v < row0 + per_rows)
            plsc.store_compressed(cd_v.at[pl.ds(off, 16)], d_v, mask=m)
            plsc.store_compressed(cs_v.at[pl.ds(off, 16)],
                                  src_v[pl.ds(b, 16)], mask=m)
            plsc.store_compressed(ct_v.at[pl.ds(off, 16)],
                                  et_v[pl.ds(b, 16)], mask=m)
            return off + jnp.sum(m.astype(jnp.int32), axis=0)

        nmine = lax.fori_loop(0, E // 16, compress, jnp.int32(0))
        ng = (nmine + 15) // 16

        def cnt_pass(g, _):
            b = g * 16
            valid = lanes < (nmine - b)
            d_v = cd_v[pl.ds(b, 16)]
            t_v = ct_v[pl.ds(b, 16)]
            rc_v = jnp.clip(d_v - row0, 0, per_rows - 1)
            for l in range(16):
                plsc.addupdate_scatter(
                    c_v, [rc_v, t_v], ones, mask=valid & (lanes == l))
            return 0

        lax.fori_loop(0, ng, cnt_pass, 0)

        for p in range(2):
            plo = row0 + p * half

            @pl.loop(0, half)
            def _(r):
                @pl.loop(0, ny, step=16)
                def _(cc):
                    acc_v.at[r, pl.ds(cc, 16)][...] = jnp.zeros(
                        (16,), jnp.float32)

            def adj_pass(g, _):
                b = g * 16
                valid = lanes < (nmine - b)
                d_v = cd_v[pl.ds(b, 16)]
                s_v = cs_v[pl.ds(b, 16)]
                in_a = valid & (d_v >= plo) & (d_v < plo + half)
                r_v = jnp.clip(d_v - plo, 0, half - 1)
                for l in range(16):
                    plsc.addupdate_scatter(
                        acc_v, [r_v, s_v], ones, mask=in_a & (lanes == l))
                return 0

            lax.fori_loop(0, ng, adj_pass, 0)
            pltpu.sync_copy(acc_v, adj_hbm.at[pl.ds(plo, half)])

        pltpu.sync_copy(c_v, cnt_hbm.at[pl.ds(row0, per_rows)])

    return k(src, dst, etype)


def _sc_edge_scores(ab, src, dst):
    E = src.shape[0]
    W = ab.shape[1]
    per_w = E // N_WORKERS
    chunk = min(per_w, 96)
    nchunks = per_w // chunk
    mesh = plsc.VectorSubcoreMesh(core_axis_name="c", subcore_axis_name="s")

    cp = pltpu.CompilerParams()
    if "needs_layout_passes" in pltpu.CompilerParams.__dataclass_fields__:
        cp = dataclasses.replace(cp, needs_layout_passes=False)

    @functools.partial(
        pl.kernel, mesh=mesh, compiler_params=cp,
        out_type=jax.ShapeDtypeStruct((E, 16), jnp.float32),
        scratch_types=[
            pltpu.VMEM((chunk,), jnp.int32),
            pltpu.VMEM((chunk,), jnp.int32),
            pltpu.VMEM((chunk, W), jnp.float32),
            pltpu.VMEM((chunk, W), jnp.float32),
            pltpu.VMEM((chunk, 16), jnp.float32),
            pltpu.SemaphoreType.DMA,
            pltpu.SemaphoreType.DMA,
        ])
    def k(ab_hbm, src_hbm, dst_hbm, out_hbm,
          src_v, dst_v, arow_v, brow_v, o_v, sem1, sem2):
        wid = lax.axis_index("s") * N_SC_CORES + lax.axis_index("c")
        base = wid * per_w
        lanes = lax.iota(jnp.int32, 16)

        @pl.loop(0, nchunks)
        def _(c):
            b = base + c * chunk
            pltpu.sync_copy(src_hbm.at[pl.ds(b, chunk)], src_v)
            pltpu.sync_copy(dst_hbm.at[pl.ds(b, chunk)], dst_v)
            cp1 = pltpu.async_copy(ab_hbm.at[src_v], arow_v, sem1)
            cp2 = pltpu.async_copy(ab_hbm.at[dst_v], brow_v, sem2)
            cp1.wait()
            cp2.wait()

            @pl.loop(0, chunk)
            def _(i):
                row = jnp.full((16,), 0, jnp.int32) + i
                a = plsc.load_gather(arow_v, [row, lanes])
                bb = plsc.load_gather(brow_v, [row, lanes + 16])
                plsc.store_scatter(o_v, [row, lanes], a + bb)

            pltpu.sync_copy(o_v, out_hbm.at[pl.ds(b, chunk)])

    return k(ab, src, dst)


def _flash_layer(s_arr, s_col, agg, wq, kv, k_col, v_col,
                 yb3, xb3, sched, D, BY=128, BX=512,
                 out_dtype=jnp.float32):
    NYr = agg.shape[0]
    NXr = kv.shape[0]
    ny = NYr // BY
    nx = NXr // BX
    scale = 1.0 / math.sqrt(float(D))
    kb_col = k_col // D
    vb_col = v_col // D
    sb_col = s_col // D

    def body(sched_ref, s_ref, a_ref, wq_ref, k_ref, v_ref,
             yb_ref, xb_ref, o_ref, h_s, q_s, acc_s, m_s, l_s):
        i = pl.program_id(0)
        j = pl.program_id(1)

        @pl.when(j == 0)
        def _():
            h = s_ref[...] + a_ref[...]
            h_s[...] = h
            q_s[...] = (jnp.dot(h.astype(jnp.bfloat16), wq_ref[...],
                                preferred_element_type=jnp.float32)
                        * scale).astype(jnp.bfloat16)
            acc_s[...] = jnp.zeros_like(acc_s)
            m_s[...] = jnp.full_like(m_s, -jnp.inf)
            l_s[...] = jnp.zeros_like(l_s)

        lo = sched_ref[0, i]
        hi = sched_ref[1, i]

        @pl.when((j >= lo) & (j <= hi))
        def _():
            sc = lax.dot_general(
                q_s[...], k_ref[...], (((1,), (1,)), ((), ())),
                preferred_element_type=jnp.float32)
            yb = yb_ref[0]
            xb = xb_ref[0]
            mask = yb == xb
            sc = jnp.where(mask, sc, NEG_INF)
            m_prev = m_s[...]
            m_new = jnp.maximum(m_prev, jnp.max(sc, axis=1, keepdims=True))
            p = jnp.exp(sc - m_new)
            corr = jnp.exp(m_prev - m_new)
            l_s[...] = l_s[...] * corr + jnp.sum(p, axis=1, keepdims=True)
            m_s[...] = m_new
            acc_s[...] = acc_s[...] * corr + jnp.dot(
                p.astype(jnp.bfloat16), v_ref[...],
                preferred_element_type=jnp.float32)

        @pl.when(j == nx - 1)
        def _():
            o_ref[...] = jnp.maximum(
                h_s[...] + acc_s[...] / l_s[...], 0.0).astype(o_ref.dtype)

    def kv_index(col):
        def f(i, j, sched):
            return (jnp.clip(j, sched[0, i], sched[1, i]), col)
        return f

    def xb_index(i, j, sched):
        return (jnp.clip(j, sched[0, i], sched[1, i]), 0, 0)

    grid_spec = pltpu.PrefetchScalarGridSpec(
        num_scalar_prefetch=1,
        grid=(ny, nx),
        in_specs=[
            pl.BlockSpec((BY, D), lambda i, j, sched: (i, sb_col)),
            pl.BlockSpec((BY, D), lambda i, j, sched: (i, 0)),
            pl.BlockSpec((D, D), lambda i, j, sched: (0, 0)),
            pl.BlockSpec((BX, D), kv_index(kb_col)),
            pl.BlockSpec((BX, D), kv_index(vb_col)),
            pl.BlockSpec((1, BY, 1), lambda i, j, sched: (i, 0, 0)),
            pl.BlockSpec((1, 1, BX), xb_index),
        ],
        out_specs=pl.BlockSpec((BY, D), lambda i, j, sched: (i, 0)),
        scratch_shapes=[
            pltpu.VMEM((BY, D), jnp.float32),
            pltpu.VMEM((BY, D), jnp.bfloat16),
            pltpu.VMEM((BY, D), jnp.float32),
            pltpu.VMEM((BY, 1), jnp.float32),
            pltpu.VMEM((BY, 1), jnp.float32),
        ],
    )
    return pl.pallas_call(
        body,
        grid_spec=grid_spec,
        out_shape=jax.ShapeDtypeStruct((NYr, D), out_dtype),
        compiler_params=pltpu.CompilerParams(
            dimension_semantics=("parallel", "arbitrary")),
    )(sched, s_arr, agg, wq, kv, kv, yb3, xb3)


def _flash_layer2(s_arr, s_col, agg, wq, kv, k_col, v_col,
                  yb3, xb2, sched, D, BY=128, BX=512,
                  out_dtype=jnp.float32):
    NYr = agg.shape[0]
    ny = NYr // BY
    scale = 1.0 / math.sqrt(float(D))
    sb_col = s_col // D
    bf16 = jnp.bfloat16

    def body(sched_ref, s_ref, a_ref, wq_ref, kv_ref, yb_ref, xb_ref, o_ref,
             kbuf, vbuf, acc_s, m_s, l_s, ksem, vsem):
        i = pl.program_id(0)
        lo = sched_ref[0, i]
        hi = sched_ref[1, i]
        h = s_ref[...] + a_ref[...]
        q = (jnp.dot(h.astype(bf16), wq_ref[...],
                     preferred_element_type=jnp.float32) * scale).astype(bf16)
        yb = yb_ref[0]

        def start_copy(jx, slot):
            pltpu.make_async_copy(
                kv_ref.at[pl.ds(jx * BX, BX), pl.ds(k_col, D)],
                kbuf.at[slot], ksem.at[slot]).start()
            pltpu.make_async_copy(
                kv_ref.at[pl.ds(jx * BX, BX), pl.ds(v_col, D)],
                vbuf.at[slot], vsem.at[slot]).start()

        start_copy(lo, 0)
        acc = jnp.zeros((BY, D), jnp.float32)
        acc_s[...] = acc
        m_s[...] = jnp.full((BY, 1), -jnp.inf, jnp.float32)
        l_s[...] = jnp.zeros((BY, 1), jnp.float32)

        def step(j, slot):
            @pl.when(j < hi)
            def _():
                start_copy(j + 1, 1 - slot)

            pltpu.make_async_copy(
                kv_ref.at[pl.ds(j * BX, BX), pl.ds(k_col, D)],
                kbuf.at[slot], ksem.at[slot]).wait()
            pltpu.make_async_copy(
                kv_ref.at[pl.ds(j * BX, BX), pl.ds(v_col, D)],
                vbuf.at[slot], vsem.at[slot]).wait()
            sc = lax.dot_general(
                q, kbuf[slot], (((1,), (1,)), ((), ())),
                preferred_element_type=jnp.float32)
            xb = xb_ref[:, pl.ds(j * BX, BX)]
            sc = jnp.where(yb == xb, sc, NEG_INF)
            m_prev = m_s[...]
            m_new = jnp.maximum(m_prev, jnp.max(sc, axis=1, keepdims=True))
            p = jnp.exp(sc - m_new)
            corr = jnp.exp(m_prev - m_new)
            l_s[...] = l_s[...] * corr + jnp.sum(p, axis=1, keepdims=True)
            m_s[...] = m_new
            acc_s[...] = acc_s[...] * corr + jnp.dot(
                p.astype(bf16), vbuf[slot],
                preferred_element_type=jnp.float32)
            return 1 - slot

        lax.fori_loop(lo, hi + 1, step, 0)
        o_ref[...] = jnp.maximum(
            h + acc_s[...] / l_s[...], 0.0).astype(o_ref.dtype)

    grid_spec = pltpu.PrefetchScalarGridSpec(
        num_scalar_prefetch=1,
        grid=(ny,),
        in_specs=[
            pl.BlockSpec((BY, D), lambda i, sched: (i, sb_col)),
            pl.BlockSpec((BY, D), lambda i, sched: (i, 0)),
            pl.BlockSpec((D, D), lambda i, sched: (0, 0)),
            pl.BlockSpec(memory_space=pl.ANY),
            pl.BlockSpec((1, BY, 1), lambda i, sched: (i, 0, 0)),
            pl.BlockSpec((1, xb2.shape[1]), lambda i, sched: (0, 0)),
        ],
        out_specs=pl.BlockSpec((BY, D), lambda i, sched: (i, 0)),
        scratch_shapes=[
            pltpu.VMEM((2, BX, D), bf16),
            pltpu.VMEM((2, BX, D), bf16),
            pltpu.VMEM((BY, D), jnp.float32),
            pltpu.VMEM((BY, 1), jnp.float32),
            pltpu.VMEM((BY, 1), jnp.float32),
            pltpu.SemaphoreType.DMA((2,)),
            pltpu.SemaphoreType.DMA((2,)),
        ],
    )
    return pl.pallas_call(
        body,
        grid_spec=grid_spec,
        out_shape=jax.ShapeDtypeStruct((NYr, D), out_dtype),
        compiler_params=pltpu.CompilerParams(
            dimension_semantics=("parallel",)),
    )(sched, s_arr, agg, wq, kv, yb3, xb2)


def _flash_layer3(y_prev, w1, A, B, cnt, embp, wq, kv, k_col, v_col,
                  yb3, xb2, sched, D, BY=256, BX=512,
                  out_dtype=jnp.float32):
    NYr = A.shape[0]
    KA = A.shape[1]
    in_d = y_prev.shape[1]
    ny = NYr // BY
    scale = 1.0 / math.sqrt(float(D))
    bf16 = jnp.bfloat16

    def body(sched_ref, y_ref, w1_ref, a_ref, b_ref, cnt_ref, embp_ref,
             wq_ref, kv_ref, yb_ref, xb_ref, o_ref,
             kbuf, vbuf, acc_s, m_s, l_s, ksem, vsem):
        i = pl.program_id(0)
        lo = sched_ref[0, i]
        hi = sched_ref[1, i]

        def start_copy(jx, slot):
            pltpu.make_async_copy(
                kv_ref.at[pl.ds(jx * BX, BX), pl.ds(k_col, D)],
                kbuf.at[slot], ksem.at[slot]).start()
            pltpu.make_async_copy(
                kv_ref.at[pl.ds(jx * BX, BX), pl.ds(v_col, D)],
                vbuf.at[slot], vsem.at[slot]).start()

        start_copy(lo, 0)
        h = (jnp.dot(y_ref[...].astype(bf16), w1_ref[...],
                     preferred_element_type=jnp.float32)
             + jnp.dot(a_ref[...].astype(bf16), b_ref[...],
                       preferred_element_type=jnp.float32)
             + jnp.dot(cnt_ref[...].astype(bf16), embp_ref[...],
                       preferred_element_type=jnp.float32))
        q = (jnp.dot(h.astype(bf16), wq_ref[...],
                     preferred_element_type=jnp.float32) * scale).astype(bf16)
        yb = yb_ref[0]

        acc_s[...] = jnp.zeros((BY, D), jnp.float32)
        m_s[...] = jnp.full((BY, 1), -jnp.inf, jnp.float32)
        l_s[...] = jnp.zeros((BY, 1), jnp.float32)

        def step(j, slot):
            @pl.when(j < hi)
            def _():
                start_copy(j + 1, 1 - slot)

            pltpu.make_async_copy(
                kv_ref.at[pl.ds(j * BX, BX), pl.ds(k_col, D)],
                kbuf.at[slot], ksem.at[slot]).wait()
            pltpu.make_async_copy(
                kv_ref.at[pl.ds(j * BX, BX), pl.ds(v_col, D)],
                vbuf.at[slot], vsem.at[slot]).wait()
            sc = lax.dot_general(
                q, kbuf[slot], (((1,), (1,)), ((), ())),
                preferred_element_type=jnp.float32)
            xb = xb_ref[:, pl.ds(j * BX, BX)]
            sc = jnp.where(yb == xb, sc, NEG_INF)
            m_prev = m_s[...]
            m_new = jnp.maximum(m_prev, jnp.max(sc, axis=1, keepdims=True))
            p = jnp.exp(sc - m_new)
            corr = jnp.exp(m_prev - m_new)
            l_s[...] = l_s[...] * corr + jnp.sum(p, axis=1, keepdims=True)
            m_s[...] = m_new
            acc_s[...] = acc_s[...] * corr + jnp.dot(
                p.astype(bf16), vbuf[slot],
                preferred_element_type=jnp.float32)
            return 1 - slot

        lax.fori_loop(lo, hi + 1, step, 0)
        o_ref[...] = jnp.maximum(
            h + acc_s[...] / l_s[...], 0.0).astype(o_ref.dtype)

    grid_spec = pltpu.PrefetchScalarGridSpec(
        num_scalar_prefetch=1,
        grid=(ny,),
        in_specs=[
            pl.BlockSpec((BY, in_d), lambda i, sched: (i, 0)),
            pl.BlockSpec((in_d, D), lambda i, sched: (0, 0)),
            pl.BlockSpec((BY, KA), lambda i, sched: (i, 0)),
            pl.BlockSpec((KA, D), lambda i, sched: (0, 0)),
            pl.BlockSpec((BY, 16), lambda i, sched: (i, 0)),
            pl.BlockSpec((16, D), lambda i, sched: (0, 0)),
            pl.BlockSpec((D, D), lambda i, sched: (0, 0)),
            pl.BlockSpec(memory_space=pl.ANY),
            pl.BlockSpec((1, BY, 1), lambda i, sched: (i, 0, 0)),
            pl.BlockSpec((1, xb2.shape[1]), lambda i, sched: (0, 0)),
        ],
        out_specs=pl.BlockSpec((BY, D), lambda i, sched: (i, 0)),
        scratch_shapes=[
            pltpu.VMEM((2, BX, D), bf16),
            pltpu.VMEM((2, BX, D), bf16),
            pltpu.VMEM((BY, D), jnp.float32),
            pltpu.VMEM((BY, 1), jnp.float32),
            pltpu.VMEM((BY, 1), jnp.float32),
            pltpu.SemaphoreType.DMA((2,)),
            pltpu.SemaphoreType.DMA((2,)),
        ],
    )
    return pl.pallas_call(
        body,
        grid_spec=grid_spec,
        out_shape=jax.ShapeDtypeStruct((NYr, D), out_dtype),
        compiler_params=pltpu.CompilerParams(
            dimension_semantics=("parallel",)),
    )(sched, y_prev, w1, A, B, cnt, embp, wq, kv, yb3, xb2)


def _block_schedule(y_batch, x_batch, BY, BX, nx):
    ny = y_batch.shape[0] // BY
    b_lo = y_batch[::BY]
    b_hi = y_batch[BY - 1::BY]
    counts = jnp.sum(
        (x_batch[None, :] == jnp.arange(NUM_BATCHES)[:, None]).astype(
            jnp.int32), axis=1)
    bounds = jnp.concatenate(
        [jnp.zeros((1,), jnp.int32), jnp.cumsum(counts)]).astype(jnp.int32)
    xs = bounds[b_lo]
    xe = bounds[b_hi + 1]
    lo = jnp.clip(xs // BX, 0, nx - 1)
    hi = jnp.clip(jnp.maximum((xe - 1) // BX, lo), 0, nx - 1)
    return jnp.stack([lo, hi]).astype(jnp.int32)


def kernel(x, x_batch, tgt_y, tgt_edge_index, tgt_edge_type, tgt_y_batch,
           params):
    p = params
    g1, g2, g3 = p['gcn1'], p['gcn2'], p['gcn3']
    src = tgt_edge_index[0]
    dst = tgt_edge_index[1]
    H1 = g1['W_self'].shape[1]
    H2 = g2['W_self'].shape[1]
    H3 = g3['W_self'].shape[1]

    BY, BX = 512, 1024
    NXr = x.shape[0]
    nx = NXr // BX
    ny = tgt_y_batch.shape[0] // BY
    sched = _block_schedule(tgt_y_batch, x_batch, BY, BX, nx)
    yb3 = tgt_y_batch.reshape(ny, BY, 1)
    xb2 = x_batch.reshape(1, NXr)

    bf16 = jnp.bfloat16

    kvw = jnp.concatenate(
        [g1['Wk'], g1['Wv'], g2['Wk'], g2['Wv'], g3['Wk'], g3['Wv']], axis=1)
    kv = _matmul(x, kvw, out_dtype=bf16, bm=2048, bn=kvw.shape[1])

    ny_nodes = tgt_y.shape[0]
    adj, cnt = _sc_build_adj(src, dst, tgt_edge_type, ny_nodes)

    def _emb_pad(emb):
        return jnp.concatenate(
            [emb.astype(bf16),
             jnp.zeros((16 - emb.shape[0], emb.shape[1]), bf16)], axis=0)

    y0 = _sc_gather_rows(p['embed'], tgt_y)
    ay0 = _matmul(adj, y0, out_dtype=bf16)
    y1 = _flash_layer3(y0, g1['W_self'].astype(bf16), ay0,
                       g1['W_nb'].astype(bf16), cnt, _emb_pad(g1['edge_emb']),
                       g1['Wq'].astype(bf16), kv, 0, H1,
                       yb3, xb2, sched, H1, BY, BX, out_dtype=bf16)

    m2 = _matmul(y1, g2['W_nb'], out_dtype=bf16)
    y2 = _flash_layer3(y1, g2['W_self'].astype(bf16), adj, m2, cnt,
                       _emb_pad(g2['edge_emb']), g2['Wq'].astype(bf16), kv,
                       2 * H1, 2 * H1 + H2, yb3, xb2, sched, H2, BY, BX,
                       out_dtype=bf16)

    m3 = _matmul(y2, g3['W_nb'], out_dtype=bf16)
    y3 = _flash_layer3(y2, g3['W_self'].astype(bf16), adj, m3, cnt,
                       _emb_pad(g3['edge_emb']), g3['Wq'].astype(bf16), kv,
                       2 * (H1 + H2), 2 * (H1 + H2) + H3,
                       yb3, xb2, sched, H3, BY, BX)

    emb_d = y3.shape[1]
    n_rel = p['Wg'].shape[1]
    y_score = _matmul(y3, p['Wz'], bias=p['bz'])
    wg_pad = jnp.concatenate(
        [jnp.pad(p['Wg'][:emb_d], ((0, 0), (0, 16 - n_rel))),
         jnp.pad(p['Wg'][emb_d:], ((0, 0), (0, 16 - n_rel))),
         jnp.zeros((emb_d, 96), jnp.float32)], axis=1)
    bg_pad = jnp.pad(p['bg'], (0, 128 - n_rel))
    ab = _matmul(y3, wg_pad, bias=bg_pad, bn=128)
    er = _sc_edge_scores(ab, src, dst)
    y_edge_rel_score = lax.slice_in_dim(er, 0, n_rel, axis=1)

    return (y3, tgt_y_batch, tgt_edge_index, tgt_edge_type, y_score,
            y_edge_rel_score)
